# Initial kernel scaffold; baseline (speedup 1.0000x reference)
import jax
import jax.numpy as jnp
from jax.experimental import pallas as pl


def kernel(inputs, ln1_scale, ln1_bias, Wqk, Wv, Wo, rot, ln2_scale, ln2_bias, W1, b1, W2, b2):
    raise NotImplementedError("write your pallas kernel here")


# speedup vs baseline: 4.3713x; 4.3713x over previous
import functools

import jax
import jax.numpy as jnp
from jax import lax
from jax.experimental import pallas as pl
from jax.experimental.pallas import tpu as pltpu
from jax.experimental.pallas import tpu_sc as plsc

B = 2
S = 4096
D = 1024
H = 16
HD = 64
MLP_D = 4096
CHUNK = 128
NB = 64
P = B * H
NCHUNK = S // CHUNK

NC = 2
NS = 16
LANES = 16
STRIDE = S // LANES
GR = 512


def _k1_body(x_ref, s1_ref, b1_ref, wqk_ref, wv_ref, rot_ref,
             qkv_ref, bkt_ref):
    x = x_ref[0]
    mu = jnp.mean(x, axis=-1, keepdims=True)
    var = jnp.mean(jnp.square(x - mu), axis=-1, keepdims=True)
    xn = (x - mu) * lax.rsqrt(var + 1e-6) * s1_ref[...] + b1_ref[...]
    qk = jnp.dot(xn, wqk_ref[...], preferred_element_type=jnp.float32)
    v = jnp.dot(xn.astype(jnp.bfloat16), wv_ref[...],
                preferred_element_type=jnp.float32)
    bs = x.shape[0]
    iota32 = lax.broadcasted_iota(jnp.int32, (bs, NB // 2), 1)
    for h in range(H):
        qh = qk[:, h * HD:(h + 1) * HD]
        qkv_ref[0, h, :, 0:HD] = qh
        qkv_ref[0, h, :, HD:2 * HD] = v[:, h * HD:(h + 1) * HD]
        r = jnp.dot(qh, rot_ref[h], preferred_element_type=jnp.float32)
        m1 = jnp.max(r, axis=-1, keepdims=True)
        i1 = jnp.min(jnp.where(r == m1, iota32, NB), axis=-1)
        m2 = jnp.max(-r, axis=-1, keepdims=True)
        i2 = jnp.min(jnp.where(-r == m2, iota32, NB), axis=-1)
        bkt_ref[0, h] = jnp.where(m1[:, 0] >= m2[:, 0], i1, NB // 2 + i2)


def _k1(x, s1, b1, wqk2, wv2, rot):
    bs = 512
    grid = (B, S // bs)
    return pl.pallas_call(
        _k1_body,
        grid=grid,
        in_specs=[
            pl.BlockSpec((1, bs, D), lambda b, s: (b, s, 0)),
            pl.BlockSpec((D,), lambda b, s: (0,)),
            pl.BlockSpec((D,), lambda b, s: (0,)),
            pl.BlockSpec((D, H * HD), lambda b, s: (0, 0)),
            pl.BlockSpec((D, H * HD), lambda b, s: (0, 0)),
            pl.BlockSpec((H, HD, NB // 2), lambda b, s: (0, 0, 0)),
        ],
        out_specs=[
            pl.BlockSpec((1, H, bs, 2 * HD), lambda b, s: (b, 0, s, 0)),
            pl.BlockSpec((1, H, bs), lambda b, s: (b, 0, s)),
        ],
        out_shape=[
            jax.ShapeDtypeStruct((B, H, S, 2 * HD), jnp.float32),
            jax.ShapeDtypeStruct((B, H, S), jnp.int32),
        ],
    )(x, s1, b1, wqk2, wv2, rot)


def _sc_sort_body(bkt_hbm, qkv_hbm,
                  skey_hbm, undo_hbm, sqkv_hbm,
                  bkt_v, cnt_v, tot_v, off_v, lr_v, sidx_v, undo_v, skey_v,
                  buf, sem):
    w = lax.axis_index("s") * NC + lax.axis_index("c")
    iot = lax.iota(jnp.int32, LANES)
    pltpu.sync_copy(bkt_hbm.at[w], bkt_v)

    def zero(i, _):
        cnt_v[pl.ds(i * LANES, LANES)] = jnp.zeros((LANES,), jnp.int32)
        return 0
    lax.fori_loop(0, NB * LANES // LANES, zero, 0)

    def p1(t, _):
        idxv = iot * STRIDE + t
        bk = plsc.load_gather(bkt_v, [idxv])
        cidx = iot * NB + bk
        c = plsc.load_gather(cnt_v, [cidx])
        plsc.store_scatter(lr_v, [idxv], c)
        plsc.store_scatter(cnt_v, [cidx], c + 1)
        return 0
    lax.fori_loop(0, STRIDE, p1, 0)

    def p2(bkt, _):
        colidx = iot * NB + bkt
        c = plsc.load_gather(cnt_v, [colidx])
        s = plsc.cumsum(c)
        plsc.store_scatter(cnt_v, [colidx], s - c)
        plsc.store_scatter(tot_v, [iot * 0 + bkt], s, mask=iot == LANES - 1)
        return 0
    lax.fori_loop(0, NB, p2, 0)

    def p3(g, carry):
        tv = tot_v[pl.ds(g * LANES, LANES)]
        s = plsc.cumsum(tv)
        off_v[pl.ds(g * LANES, LANES)] = s - tv + carry
        return carry + jnp.sum(tv)
    lax.fori_loop(0, NB // LANES, p3, jnp.int32(0))

    def p4(t, _):
        idxv = iot * STRIDE + t
        bk = plsc.load_gather(bkt_v, [idxv])
        lr = plsc.load_gather(lr_v, [idxv])
        sp = plsc.load_gather(cnt_v, [iot * NB + bk])
        ob = plsc.load_gather(off_v, [bk])
        pos = ob + sp + lr
        plsc.store_scatter(sidx_v, [pos], idxv)
        plsc.store_scatter(undo_v, [idxv], pos)
        plsc.store_scatter(skey_v, [pos], bk * S + idxv)
        return 0
    lax.fori_loop(0, STRIDE, p4, 0)

    pltpu.sync_copy(skey_v, skey_hbm.at[w])
    pltpu.sync_copy(undo_v, undo_hbm.at[w])

    for j in range(S // GR):
        idxs = sidx_v.at[pl.ds(j * GR, GR)]
        pltpu.async_copy(qkv_hbm.at[w].at[idxs], buf, sem).wait()
        pltpu.sync_copy(buf, sqkv_hbm.at[w].at[pl.ds(j * GR, GR)])


def _k2(bkt, qkv):
    mesh = plsc.VectorSubcoreMesh(core_axis_name="c", subcore_axis_name="s",
                                  num_cores=NC)
    f = functools.partial(
        pl.kernel,
        out_type=(
            jax.ShapeDtypeStruct((P, S), jnp.int32),
            jax.ShapeDtypeStruct((P, S), jnp.int32),
            jax.ShapeDtypeStruct((P, S, 2 * HD), jnp.float32),
        ),
        mesh=mesh,
        scratch_types=[
            pltpu.VMEM((S,), jnp.int32),
            pltpu.VMEM((NB * LANES,), jnp.int32),
            pltpu.VMEM((NB,), jnp.int32),
            pltpu.VMEM((NB,), jnp.int32),
            pltpu.VMEM((S,), jnp.int32),
            pltpu.VMEM((S,), jnp.int32),
            pltpu.VMEM((S,), jnp.int32),
            pltpu.VMEM((S,), jnp.int32),
            pltpu.VMEM((GR, 2 * HD), jnp.float32),
            pltpu.SemaphoreType.DMA,
        ],
        compiler_params=pltpu.CompilerParams(needs_layout_passes=False),
    )(_sc_sort_body)
    return f(bkt, qkv)


def _sc_unsort_body(undo_hbm, os_hbm, attn_hbm, undo_v, buf, sem):
    w = lax.axis_index("s") * NC + lax.axis_index("c")
    pltpu.sync_copy(undo_hbm.at[w], undo_v)
    for j in range(S // GR):
        idxs = undo_v.at[pl.ds(j * GR, GR)]
        pltpu.async_copy(os_hbm.at[w].at[idxs], buf, sem).wait()
        pltpu.sync_copy(buf, attn_hbm.at[w].at[pl.ds(j * GR, GR)])


def _k4(undo, osort):
    mesh = plsc.VectorSubcoreMesh(core_axis_name="c", subcore_axis_name="s",
                                  num_cores=NC)
    f = functools.partial(
        pl.kernel,
        out_type=jax.ShapeDtypeStruct((P, S, 2 * HD), jnp.float32),
        mesh=mesh,
        scratch_types=[
            pltpu.VMEM((S,), jnp.int32),
            pltpu.VMEM((GR, 2 * HD), jnp.float32),
            pltpu.SemaphoreType.DMA,
        ],
        compiler_params=pltpu.CompilerParams(needs_layout_passes=False),
    )(_sc_unsort_body)
    return f(undo, osort)


def _k3_body(qv_ref, pv_ref, nv_ref, tq_ref, tp_ref, tn_ref, o_ref):
    blk = qv_ref[0]
    q = blk[:, :HD]
    kcat = jnp.concatenate([pv_ref[0, :, :HD], q, nv_ref[0, :, :HD]], axis=0)
    knorm = kcat / (jnp.sqrt(jnp.sum(jnp.square(kcat), axis=-1,
                                     keepdims=True)) + 1e-6)
    vcat = jnp.concatenate([pv_ref[0, :, HD:], blk[:, HD:], nv_ref[0, :, HD:]],
                           axis=0)
    dots = jax.lax.dot_general(
        q.astype(jnp.bfloat16), knorm.astype(jnp.bfloat16),
        (((1,), (1,)), ((), ())),
        preferred_element_type=jnp.float32) * (1.0 / 8.0)
    tq = tq_ref[0, 0]
    tk = jnp.concatenate([tp_ref[0, 0], tq, tn_ref[0, 0]])
    qb = tq // S
    kb = tk // S
    dots = jnp.where(qb[:, None] != kb[None, :], dots - 1e9, dots)
    dots = jnp.where(tq[:, None] == tk[None, :], dots - 1e5, dots)
    m = jnp.max(dots, axis=-1, keepdims=True)
    e = jnp.exp(dots - m)
    wgt = e / jnp.sum(e, axis=-1, keepdims=True)
    o = jnp.dot(wgt.astype(jnp.bfloat16), vcat.astype(jnp.bfloat16),
                preferred_element_type=jnp.float32)
    o_ref[0] = jnp.concatenate([o, jnp.zeros_like(o)], axis=1)


def _k3(sqkv, skey3):
    def prev(p, c):
        return (p, (c + NCHUNK - 1) % NCHUNK, 0)

    def nxt(p, c):
        return (p, (c + 1) % NCHUNK, 0)

    def tprev(p, c):
        return (p * NCHUNK + (c + NCHUNK - 1) % NCHUNK, 0, 0)

    def tnxt(p, c):
        return (p * NCHUNK + (c + 1) % NCHUNK, 0, 0)

    return pl.pallas_call(
        _k3_body,
        grid=(P, NCHUNK),
        in_specs=[
            pl.BlockSpec((1, CHUNK, 2 * HD), lambda p, c: (p, c, 0)),
            pl.BlockSpec((1, CHUNK, 2 * HD), prev),
            pl.BlockSpec((1, CHUNK, 2 * HD), nxt),
            pl.BlockSpec((1, 1, CHUNK), lambda p, c: (p * NCHUNK + c, 0, 0)),
            pl.BlockSpec((1, 1, CHUNK), tprev),
            pl.BlockSpec((1, 1, CHUNK), tnxt),
        ],
        out_specs=pl.BlockSpec((1, CHUNK, 2 * HD), lambda p, c: (p, c, 0)),
        out_shape=jax.ShapeDtypeStruct((P, S, 2 * HD), jnp.float32),
    )(sqkv, sqkv, sqkv, skey3, skey3, skey3)


def _k5_body(a_ref, x_ref, wo_ref, s2_ref, b2_ref, x2_ref, yn_ref):
    cat = jnp.concatenate([a_ref[h, :, :HD] for h in range(H)], axis=1)
    out = jnp.dot(cat.astype(jnp.bfloat16), wo_ref[...],
                  preferred_element_type=jnp.float32)
    x2 = x_ref[0] + out
    x2_ref[0] = x2
    mu = jnp.mean(x2, axis=-1, keepdims=True)
    var = jnp.mean(jnp.square(x2 - mu), axis=-1, keepdims=True)
    yn = (x2 - mu) * lax.rsqrt(var + 1e-6) * s2_ref[...] + b2_ref[...]
    yn_ref[0] = yn.astype(jnp.bfloat16)


def _k5(attn, x, wo2, s2, b2):
    bs = 512
    return pl.pallas_call(
        _k5_body,
        grid=(B, S // bs),
        in_specs=[
            pl.BlockSpec((H, bs, 2 * HD), lambda b, s: (b, s, 0)),
            pl.BlockSpec((1, bs, D), lambda b, s: (b, s, 0)),
            pl.BlockSpec((H * HD, D), lambda b, s: (0, 0)),
            pl.BlockSpec((D,), lambda b, s: (0,)),
            pl.BlockSpec((D,), lambda b, s: (0,)),
        ],
        out_specs=[
            pl.BlockSpec((1, bs, D), lambda b, s: (b, s, 0)),
            pl.BlockSpec((1, bs, D), lambda b, s: (b, s, 0)),
        ],
        out_shape=[
            jax.ShapeDtypeStruct((B, S, D), jnp.float32),
            jax.ShapeDtypeStruct((B, S, D), jnp.bfloat16),
        ],
    )(attn, x, wo2, s2, b2)


def _k6_body(yn_ref, w1_ref, b1_ref, w2_ref, x2_ref, b2_ref, out_ref,
             acc_ref):
    n = pl.program_id(1)
    nblocks = pl.num_programs(1)

    @pl.when(n == 0)
    def _():
        acc_ref[...] = jnp.zeros_like(acc_ref)

    hid = jnp.dot(yn_ref[...], w1_ref[...],
                  preferred_element_type=jnp.float32) + b1_ref[...]
    hid = jnp.maximum(hid, 0.0).astype(jnp.bfloat16)
    acc_ref[...] += jnp.dot(hid, w2_ref[...],
                            preferred_element_type=jnp.float32)

    @pl.when(n == nblocks - 1)
    def _():
        out_ref[...] = acc_ref[...] + x2_ref[...] + b2_ref[...]


def _k6(yn2, w1, b1, w2, x22, b2):
    rb = 512
    nb = 1024
    rows = B * S
    return pl.pallas_call(
        _k6_body,
        grid=(rows // rb, MLP_D // nb),
        in_specs=[
            pl.BlockSpec((rb, D), lambda r, n: (r, 0)),
            pl.BlockSpec((D, nb), lambda r, n: (0, n)),
            pl.BlockSpec((nb,), lambda r, n: (n,)),
            pl.BlockSpec((nb, D), lambda r, n: (n, 0)),
            pl.BlockSpec((rb, D), lambda r, n: (r, 0)),
            pl.BlockSpec((D,), lambda r, n: (0,)),
        ],
        out_specs=pl.BlockSpec((rb, D), lambda r, n: (r, 0)),
        out_shape=jax.ShapeDtypeStruct((rows, D), jnp.float32),
        scratch_shapes=[pltpu.VMEM((rb, D), jnp.float32)],
    )(yn2, w1, b1, w2, x22, b2)


def kernel(inputs, ln1_scale, ln1_bias, Wqk, Wv, Wo, rot, ln2_scale,
           ln2_bias, W1, b1, W2, b2):
    wqk2 = Wqk.reshape(D, H * HD)
    wv2 = Wv.reshape(D, H * HD).astype(jnp.bfloat16)
    wo2 = Wo.reshape(H * HD, D).astype(jnp.bfloat16)

    qkv4, bkt3 = _k1(inputs, ln1_scale, ln1_bias, wqk2, wv2, rot)
    qkv = qkv4.reshape(P, S, 2 * HD)
    bkt = bkt3.reshape(P, S)

    skey, undo, sqkv = _k2(bkt, qkv)
    skey3 = skey.reshape(P * NCHUNK, 1, CHUNK)

    osort = _k3(sqkv, skey3)
    attn = _k4(undo, osort)

    x2, yn = _k5(attn, inputs, wo2, ln2_scale, ln2_bias)

    final = _k6(yn.reshape(B * S, D), W1.astype(jnp.bfloat16), b1,
                W2.astype(jnp.bfloat16), x2.reshape(B * S, D), b2)
    return final.reshape(B, S, D)


# speedup vs baseline: 6.1999x; 1.4183x over previous
import functools

import jax
import jax.numpy as jnp
from jax import lax
from jax.experimental import pallas as pl
from jax.experimental.pallas import tpu as pltpu
from jax.experimental.pallas import tpu_sc as plsc

B = 2
S = 4096
D = 1024
H = 16
HD = 64
MLP_D = 4096
CHUNK = 128
NB = 64
P = B * H
NCHUNK = S // CHUNK

NC = 2
NS = 16
LANES = 16
STRIDE = S // LANES
GR = 512


def _k1_body(x_ref, s1_ref, b1_ref, wqk_ref, wv_ref, rot_ref,
             qkv_ref, bkt_ref, rn_ref):
    x = x_ref[0]
    mu = jnp.mean(x, axis=-1, keepdims=True)
    var = jnp.mean(jnp.square(x - mu), axis=-1, keepdims=True)
    xn = (x - mu) * lax.rsqrt(var + 1e-6) * s1_ref[...] + b1_ref[...]
    qk = jnp.dot(xn, wqk_ref[...], preferred_element_type=jnp.float32)
    v = jnp.dot(xn.astype(jnp.bfloat16), wv_ref[...],
                preferred_element_type=jnp.float32)
    bs = x.shape[0]
    iota32 = lax.broadcasted_iota(jnp.int32, (bs, NB // 2), 1)
    for h in range(H):
        qh = qk[:, h * HD:(h + 1) * HD]
        qkv_ref[0, h, :, 0:HD] = qh
        qkv_ref[0, h, :, HD:2 * HD] = v[:, h * HD:(h + 1) * HD]
        ss = jnp.sum(jnp.square(qh), axis=-1)
        rn_ref[0, h] = 0.125 / (jnp.sqrt(ss) + 1e-6)
        r = jnp.dot(qh, rot_ref[h], preferred_element_type=jnp.float32)
        m1 = jnp.max(r, axis=-1, keepdims=True)
        i1 = jnp.min(jnp.where(r == m1, iota32, NB), axis=-1)
        m2 = jnp.max(-r, axis=-1, keepdims=True)
        i2 = jnp.min(jnp.where(-r == m2, iota32, NB), axis=-1)
        bkt_ref[0, h] = jnp.where(m1[:, 0] >= m2[:, 0], i1, NB // 2 + i2)


def _k1(x, s1, b1, wqk2, wv2, rot):
    bs = 512
    grid = (B, S // bs)
    return pl.pallas_call(
        _k1_body,
        grid=grid,
        in_specs=[
            pl.BlockSpec((1, bs, D), lambda b, s: (b, s, 0)),
            pl.BlockSpec((D,), lambda b, s: (0,)),
            pl.BlockSpec((D,), lambda b, s: (0,)),
            pl.BlockSpec((D, H * HD), lambda b, s: (0, 0)),
            pl.BlockSpec((D, H * HD), lambda b, s: (0, 0)),
            pl.BlockSpec((H, HD, NB // 2), lambda b, s: (0, 0, 0)),
        ],
        out_specs=[
            pl.BlockSpec((1, H, bs, 2 * HD), lambda b, s: (b, 0, s, 0)),
            pl.BlockSpec((1, H, bs), lambda b, s: (b, 0, s)),
            pl.BlockSpec((1, H, bs), lambda b, s: (b, 0, s)),
        ],
        out_shape=[
            jax.ShapeDtypeStruct((B, H, S, 2 * HD), jnp.float32),
            jax.ShapeDtypeStruct((B, H, S), jnp.int32),
            jax.ShapeDtypeStruct((B, H, S), jnp.float32),
        ],
    )(x, s1, b1, wqk2, wv2, rot)


def _sc_sort_body(bkt_hbm, rn_hbm, qkv_hbm,
                  skey_hbm, undo_hbm, srn_hbm, sqkv_hbm,
                  bkt_v, cnt_v, tot_v, off_v, lr_v, sidx_v, undo_v, skey_v,
                  rn_v, srn_v, buf, sem):
    w = lax.axis_index("s") * NC + lax.axis_index("c")
    iot = lax.iota(jnp.int32, LANES)
    pltpu.sync_copy(bkt_hbm.at[w], bkt_v)
    pltpu.sync_copy(rn_hbm.at[w], rn_v)

    def zero(i, _):
        cnt_v[pl.ds(i * LANES, LANES)] = jnp.zeros((LANES,), jnp.int32)
        return 0
    lax.fori_loop(0, NB * LANES // LANES, zero, 0)

    def p1(t, _):
        idxv = iot * STRIDE + t
        bk = plsc.load_gather(bkt_v, [idxv])
        cidx = iot * NB + bk
        c = plsc.load_gather(cnt_v, [cidx])
        plsc.store_scatter(lr_v, [idxv], c)
        plsc.store_scatter(cnt_v, [cidx], c + 1)
        return 0
    lax.fori_loop(0, STRIDE, p1, 0)

    def p2(bkt, _):
        colidx = iot * NB + bkt
        c = plsc.load_gather(cnt_v, [colidx])
        s = plsc.cumsum(c)
        plsc.store_scatter(cnt_v, [colidx], s - c)
        plsc.store_scatter(tot_v, [iot * 0 + bkt], s, mask=iot == LANES - 1)
        return 0
    lax.fori_loop(0, NB, p2, 0)

    def p3(g, carry):
        tv = tot_v[pl.ds(g * LANES, LANES)]
        s = plsc.cumsum(tv)
        off_v[pl.ds(g * LANES, LANES)] = s - tv + carry
        return carry + jnp.sum(tv)
    lax.fori_loop(0, NB // LANES, p3, jnp.int32(0))

    def p4(t, _):
        idxv = iot * STRIDE + t
        bk = plsc.load_gather(bkt_v, [idxv])
        lr = plsc.load_gather(lr_v, [idxv])
        sp = plsc.load_gather(cnt_v, [iot * NB + bk])
        ob = plsc.load_gather(off_v, [bk])
        pos = ob + sp + lr
        plsc.store_scatter(sidx_v, [pos], idxv)
        plsc.store_scatter(undo_v, [idxv], pos)
        plsc.store_scatter(skey_v, [pos], bk * S + idxv)
        rv = plsc.load_gather(rn_v, [idxv])
        plsc.store_scatter(srn_v, [pos], rv)
        return 0
    lax.fori_loop(0, STRIDE, p4, 0)

    pltpu.sync_copy(skey_v, skey_hbm.at[w])
    pltpu.sync_copy(undo_v, undo_hbm.at[w])
    pltpu.sync_copy(srn_v, srn_hbm.at[w])

    for j in range(S // GR):
        idxs = sidx_v.at[pl.ds(j * GR, GR)]
        pltpu.async_copy(qkv_hbm.at[w].at[idxs], buf, sem).wait()
        pltpu.sync_copy(buf, sqkv_hbm.at[w].at[pl.ds(j * GR, GR)])


def _k2(bkt, rn, qkv):
    mesh = plsc.VectorSubcoreMesh(core_axis_name="c", subcore_axis_name="s",
                                  num_cores=NC)
    f = functools.partial(
        pl.kernel,
        out_type=(
            jax.ShapeDtypeStruct((P, S), jnp.int32),
            jax.ShapeDtypeStruct((P, S), jnp.int32),
            jax.ShapeDtypeStruct((P, S), jnp.float32),
            jax.ShapeDtypeStruct((P, S, 2 * HD), jnp.float32),
        ),
        mesh=mesh,
        scratch_types=[
            pltpu.VMEM((S,), jnp.int32),
            pltpu.VMEM((NB * LANES,), jnp.int32),
            pltpu.VMEM((NB,), jnp.int32),
            pltpu.VMEM((NB,), jnp.int32),
            pltpu.VMEM((S,), jnp.int32),
            pltpu.VMEM((S,), jnp.int32),
            pltpu.VMEM((S,), jnp.int32),
            pltpu.VMEM((S,), jnp.int32),
            pltpu.VMEM((S,), jnp.float32),
            pltpu.VMEM((S,), jnp.float32),
            pltpu.VMEM((GR, 2 * HD), jnp.float32),
            pltpu.SemaphoreType.DMA,
        ],
        compiler_params=pltpu.CompilerParams(needs_layout_passes=False),
    )(_sc_sort_body)
    return f(bkt, rn, qkv)


def _sc_unsort_body(undo_hbm, os_hbm, attn_hbm, undo_v, buf, sem):
    w = lax.axis_index("s") * NC + lax.axis_index("c")
    pltpu.sync_copy(undo_hbm.at[w], undo_v)
    for j in range(S // GR):
        idxs = undo_v.at[pl.ds(j * GR, GR)]
        pltpu.async_copy(os_hbm.at[w].at[idxs], buf, sem).wait()
        pltpu.sync_copy(buf, attn_hbm.at[w].at[pl.ds(j * GR, GR)])


def _k4(undo, osort):
    mesh = plsc.VectorSubcoreMesh(core_axis_name="c", subcore_axis_name="s",
                                  num_cores=NC)
    f = functools.partial(
        pl.kernel,
        out_type=jax.ShapeDtypeStruct((P, S, 2 * HD), jnp.float32),
        mesh=mesh,
        scratch_types=[
            pltpu.VMEM((S,), jnp.int32),
            pltpu.VMEM((GR, 2 * HD), jnp.float32),
            pltpu.SemaphoreType.DMA,
        ],
        compiler_params=pltpu.CompilerParams(needs_layout_passes=False),
    )(_sc_unsort_body)
    return f(undo, osort)


CPB = 8
KBS = CPB * CHUNK
NSB = S // KBS


def _k3_body(sb_ref, pb_ref, nb_ref, ts_ref, tp_ref, tn_ref,
             rs_ref, rp_ref, rn_ref, o_ref):
    sblk = sb_ref[0]
    tsel = ts_ref[0, 0]
    rsel = rs_ref[0, 0]
    eye = (lax.broadcasted_iota(jnp.int32, (CHUNK, CHUNK), 0)
           == lax.broadcasted_iota(jnp.int32, (CHUNK, CHUNK), 1))
    outs = []
    for i in range(CPB):
        lo = i * CHUNK
        q = sblk[lo:lo + CHUNK, :HD]
        qb = (tsel[lo:lo + CHUNK] // S)[:, None]
        qbf = q.astype(jnp.bfloat16)
        if i == 0:
            kvp = pb_ref[0]
            tkp = tp_ref[0, 0, (CPB - 1) * CHUNK:]
            rkp = rp_ref[0, 0, (CPB - 1) * CHUNK:]
        else:
            kvp = sblk[lo - CHUNK:lo, :]
            tkp = tsel[lo - CHUNK:lo]
            rkp = rsel[lo - CHUNK:lo]
        kvs = sblk[lo:lo + CHUNK, :]
        tks = tsel[lo:lo + CHUNK]
        rks = rsel[lo:lo + CHUNK]
        if i == CPB - 1:
            kvn = nb_ref[0]
            tkn = tn_ref[0, 0, :CHUNK]
            rkn = rn_ref[0, 0, :CHUNK]
        else:
            kvn = sblk[lo + CHUNK:lo + 2 * CHUNK, :]
            tkn = tsel[lo + CHUNK:lo + 2 * CHUNK]
            rkn = rsel[lo + CHUNK:lo + 2 * CHUNK]
        ds = []
        for j, (kv, tk, rk) in enumerate(
                ((kvp, tkp, rkp), (kvs, tks, rks), (kvn, tkn, rkn))):
            d = jax.lax.dot_general(
                qbf, kv[:, :HD].astype(jnp.bfloat16),
                (((1,), (1,)), ((), ())),
                preferred_element_type=jnp.float32) * rk[None, :]
            d = jnp.where(qb != (tk // S)[None, :], d - 1e9, d)
            if j == 1:
                d = jnp.where(eye, d - 1e5, d)
            ds.append(d)
        m = jnp.maximum(jnp.max(ds[0], axis=-1, keepdims=True),
                        jnp.maximum(jnp.max(ds[1], axis=-1, keepdims=True),
                                    jnp.max(ds[2], axis=-1, keepdims=True)))
        es = [jnp.exp(d - m) for d in ds]
        tot = (jnp.sum(es[0], axis=-1, keepdims=True)
               + jnp.sum(es[1], axis=-1, keepdims=True)
               + jnp.sum(es[2], axis=-1, keepdims=True))
        o = (jnp.dot(es[0].astype(jnp.bfloat16), kvp[:, HD:].astype(jnp.bfloat16),
                     preferred_element_type=jnp.float32)
             + jnp.dot(es[1].astype(jnp.bfloat16), kvs[:, HD:].astype(jnp.bfloat16),
                       preferred_element_type=jnp.float32)
             + jnp.dot(es[2].astype(jnp.bfloat16), kvn[:, HD:].astype(jnp.bfloat16),
                       preferred_element_type=jnp.float32))
        o = o * (1.0 / tot)
        outs.append(jnp.concatenate([o, jnp.zeros_like(o)], axis=1))
    o_ref[0] = jnp.concatenate(outs, axis=0)


def _k3(sqkv, skey3, srn3):
    def prev(p, c):
        return (p, (c * CPB + NCHUNK - 1) % NCHUNK, 0)

    def nxt(p, c):
        return (p, (c * CPB + CPB) % NCHUNK, 0)

    def tprev(p, c):
        return (p * NSB + (c + NSB - 1) % NSB, 0, 0)

    def tnxt(p, c):
        return (p * NSB + (c + 1) % NSB, 0, 0)

    def tself(p, c):
        return (p * NSB + c, 0, 0)

    return pl.pallas_call(
        _k3_body,
        grid=(P, NSB),
        in_specs=[
            pl.BlockSpec((1, KBS, 2 * HD), lambda p, c: (p, c, 0)),
            pl.BlockSpec((1, CHUNK, 2 * HD), prev),
            pl.BlockSpec((1, CHUNK, 2 * HD), nxt),
            pl.BlockSpec((1, 1, KBS), tself),
            pl.BlockSpec((1, 1, KBS), tprev),
            pl.BlockSpec((1, 1, KBS), tnxt),
            pl.BlockSpec((1, 1, KBS), tself),
            pl.BlockSpec((1, 1, KBS), tprev),
            pl.BlockSpec((1, 1, KBS), tnxt),
        ],
        out_specs=pl.BlockSpec((1, KBS, 2 * HD), lambda p, c: (p, c, 0)),
        out_shape=jax.ShapeDtypeStruct((P, S, 2 * HD), jnp.float32),
    )(sqkv, sqkv, sqkv, skey3, skey3, skey3, srn3, srn3, srn3)


def _k5_body(a_ref, x_ref, wo_ref, s2_ref, b2_ref, x2_ref, yn_ref):
    cat = jnp.concatenate([a_ref[h, :, :HD] for h in range(H)], axis=1)
    out = jnp.dot(cat.astype(jnp.bfloat16), wo_ref[...],
                  preferred_element_type=jnp.float32)
    x2 = x_ref[0] + out
    x2_ref[0] = x2
    mu = jnp.mean(x2, axis=-1, keepdims=True)
    var = jnp.mean(jnp.square(x2 - mu), axis=-1, keepdims=True)
    yn = (x2 - mu) * lax.rsqrt(var + 1e-6) * s2_ref[...] + b2_ref[...]
    yn_ref[0] = yn.astype(jnp.bfloat16)


def _k5(attn, x, wo2, s2, b2):
    bs = 512
    return pl.pallas_call(
        _k5_body,
        grid=(B, S // bs),
        in_specs=[
            pl.BlockSpec((H, bs, 2 * HD), lambda b, s: (b, s, 0)),
            pl.BlockSpec((1, bs, D), lambda b, s: (b, s, 0)),
            pl.BlockSpec((H * HD, D), lambda b, s: (0, 0)),
            pl.BlockSpec((D,), lambda b, s: (0,)),
            pl.BlockSpec((D,), lambda b, s: (0,)),
        ],
        out_specs=[
            pl.BlockSpec((1, bs, D), lambda b, s: (b, s, 0)),
            pl.BlockSpec((1, bs, D), lambda b, s: (b, s, 0)),
        ],
        out_shape=[
            jax.ShapeDtypeStruct((B, S, D), jnp.float32),
            jax.ShapeDtypeStruct((B, S, D), jnp.bfloat16),
        ],
    )(attn, x, wo2, s2, b2)


def _k6_body(yn_ref, w1_ref, b1_ref, w2_ref, x2_ref, b2_ref, out_ref,
             acc_ref):
    n = pl.program_id(1)
    nblocks = pl.num_programs(1)

    @pl.when(n == 0)
    def _():
        acc_ref[...] = jnp.zeros_like(acc_ref)

    hid = jnp.dot(yn_ref[...], w1_ref[...],
                  preferred_element_type=jnp.float32) + b1_ref[...]
    hid = jnp.maximum(hid, 0.0).astype(jnp.bfloat16)
    acc_ref[...] += jnp.dot(hid, w2_ref[...],
                            preferred_element_type=jnp.float32)

    @pl.when(n == nblocks - 1)
    def _():
        out_ref[...] = acc_ref[...] + x2_ref[...] + b2_ref[...]


def _k6(yn2, w1, b1, w2, x22, b2):
    rb = 512
    nb = 1024
    rows = B * S
    return pl.pallas_call(
        _k6_body,
        grid=(rows // rb, MLP_D // nb),
        in_specs=[
            pl.BlockSpec((rb, D), lambda r, n: (r, 0)),
            pl.BlockSpec((D, nb), lambda r, n: (0, n)),
            pl.BlockSpec((nb,), lambda r, n: (n,)),
            pl.BlockSpec((nb, D), lambda r, n: (n, 0)),
            pl.BlockSpec((rb, D), lambda r, n: (r, 0)),
            pl.BlockSpec((D,), lambda r, n: (0,)),
        ],
        out_specs=pl.BlockSpec((rb, D), lambda r, n: (r, 0)),
        out_shape=jax.ShapeDtypeStruct((rows, D), jnp.float32),
        scratch_shapes=[pltpu.VMEM((rb, D), jnp.float32)],
    )(yn2, w1, b1, w2, x22, b2)


def kernel(inputs, ln1_scale, ln1_bias, Wqk, Wv, Wo, rot, ln2_scale,
           ln2_bias, W1, b1, W2, b2):
    wqk2 = Wqk.reshape(D, H * HD)
    wv2 = Wv.reshape(D, H * HD).astype(jnp.bfloat16)
    wo2 = Wo.reshape(H * HD, D).astype(jnp.bfloat16)

    qkv4, bkt3, rn4 = _k1(inputs, ln1_scale, ln1_bias, wqk2, wv2, rot)
    qkv = qkv4.reshape(P, S, 2 * HD)
    bkt = bkt3.reshape(P, S)
    rn = rn4.reshape(P, S)

    skey, undo, srn, sqkv = _k2(bkt, rn, qkv)
    skey3 = skey.reshape(P * NSB, 1, KBS)
    srn3 = srn.reshape(P * NSB, 1, KBS)

    osort = _k3(sqkv, skey3, srn3)
    attn = _k4(undo, osort)

    x2, yn = _k5(attn, inputs, wo2, ln2_scale, ln2_bias)

    final = _k6(yn.reshape(B * S, D), W1.astype(jnp.bfloat16), b1,
                W2.astype(jnp.bfloat16), x2.reshape(B * S, D), b2)
    return final.reshape(B, S, D)


# speedup vs baseline: 6.6881x; 1.0787x over previous
import functools

import jax
import jax.numpy as jnp
from jax import lax
from jax.experimental import pallas as pl
from jax.experimental.pallas import tpu as pltpu
from jax.experimental.pallas import tpu_sc as plsc

B = 2
S = 4096
D = 1024
H = 16
HD = 64
MLP_D = 4096
CHUNK = 128
NB = 64
P = B * H
NCHUNK = S // CHUNK

NC = 2
NS = 16
LANES = 16
STRIDE = S // LANES
GR = 256


def _k1_body(x_ref, s1_ref, b1_ref, wqk_ref, wv_ref, rot_ref,
             qkv_ref, bkt_ref, rn_ref):
    x = x_ref[0]
    mu = jnp.mean(x, axis=-1, keepdims=True)
    var = jnp.mean(jnp.square(x - mu), axis=-1, keepdims=True)
    xn = (x - mu) * lax.rsqrt(var + 1e-6) * s1_ref[...] + b1_ref[...]
    qk = jnp.dot(xn, wqk_ref[...], preferred_element_type=jnp.float32)
    v = jnp.dot(xn.astype(jnp.bfloat16), wv_ref[...],
                preferred_element_type=jnp.float32)
    bs = x.shape[0]
    rall = jnp.dot(qk, rot_ref[...], preferred_element_type=jnp.float32)
    aabs = jnp.abs(rall)
    codes = (lax.broadcasted_iota(jnp.int32, (bs, H * NB // 2), 1) % (NB // 2)
             + jnp.where(rall < 0.0, NB // 2, 0))
    for h in range(H):
        qh = qk[:, h * HD:(h + 1) * HD]
        qkv_ref[0, h, :, 0:HD] = qh
        qkv_ref[0, h, :, HD:2 * HD] = v[:, h * HD:(h + 1) * HD]
        ss = jnp.sum(jnp.square(qh), axis=-1)
        rn_ref[0, h] = 0.125 / (jnp.sqrt(ss) + 1e-6)
        a = aabs[:, h * (NB // 2):(h + 1) * (NB // 2)]
        m = jnp.max(a, axis=-1, keepdims=True)
        cand = jnp.where(a == m, codes[:, h * (NB // 2):(h + 1) * (NB // 2)],
                         NB)
        bkt_ref[0, h] = jnp.min(cand, axis=-1)


def _k1(x, s1, b1, wqk2, wv2, rot):
    bs = 512
    grid = (B, S // bs)
    return pl.pallas_call(
        _k1_body,
        grid=grid,
        in_specs=[
            pl.BlockSpec((1, bs, D), lambda b, s: (b, s, 0)),
            pl.BlockSpec((D,), lambda b, s: (0,)),
            pl.BlockSpec((D,), lambda b, s: (0,)),
            pl.BlockSpec((D, H * HD), lambda b, s: (0, 0)),
            pl.BlockSpec((D, H * HD), lambda b, s: (0, 0)),
            pl.BlockSpec((H * HD, H * NB // 2), lambda b, s: (0, 0)),
        ],
        out_specs=[
            pl.BlockSpec((1, H, bs, 2 * HD), lambda b, s: (b, 0, s, 0)),
            pl.BlockSpec((1, H, bs), lambda b, s: (b, 0, s)),
            pl.BlockSpec((1, H, bs), lambda b, s: (b, 0, s)),
        ],
        out_shape=[
            jax.ShapeDtypeStruct((B, H, S, 2 * HD), jnp.float32),
            jax.ShapeDtypeStruct((B, H, S), jnp.int32),
            jax.ShapeDtypeStruct((B, H, S), jnp.float32),
        ],
    )(x, s1, b1, wqk2, wv2, rot)


def _sc_sort_body(bkt_hbm, rn_hbm, qkv_hbm,
                  skey_hbm, undo_hbm, srn_hbm, sqkv_hbm,
                  bkt_v, cnt_v, tot_v, off_v, lr_v, sidx_v, undo_v, skey_v,
                  rn_v, srn_v, buf, buf2, sem, sem2):
    w = lax.axis_index("s") * NC + lax.axis_index("c")
    iot = lax.iota(jnp.int32, LANES)
    pltpu.sync_copy(bkt_hbm.at[w], bkt_v)
    pltpu.sync_copy(rn_hbm.at[w], rn_v)

    def zero(i, _):
        cnt_v[pl.ds(i * LANES, LANES)] = jnp.zeros((LANES,), jnp.int32)
        return 0
    lax.fori_loop(0, NB * LANES // LANES, zero, 0)

    def p1(t, _):
        idxv = iot * STRIDE + t
        bk = plsc.load_gather(bkt_v, [idxv])
        cidx = iot * NB + bk
        c = plsc.load_gather(cnt_v, [cidx])
        plsc.store_scatter(lr_v, [idxv], c)
        plsc.store_scatter(cnt_v, [cidx], c + 1)
        return 0
    lax.fori_loop(0, STRIDE, p1, 0)

    def p2(bkt, _):
        colidx = iot * NB + bkt
        c = plsc.load_gather(cnt_v, [colidx])
        s = plsc.cumsum(c)
        plsc.store_scatter(cnt_v, [colidx], s - c)
        plsc.store_scatter(tot_v, [iot * 0 + bkt], s, mask=iot == LANES - 1)
        return 0
    lax.fori_loop(0, NB, p2, 0)

    def p3(g, carry):
        tv = tot_v[pl.ds(g * LANES, LANES)]
        s = plsc.cumsum(tv)
        off_v[pl.ds(g * LANES, LANES)] = s - tv + carry
        return carry + jnp.sum(tv)
    lax.fori_loop(0, NB // LANES, p3, jnp.int32(0))

    def p4(t, _):
        idxv = iot * STRIDE + t
        bk = plsc.load_gather(bkt_v, [idxv])
        lr = plsc.load_gather(lr_v, [idxv])
        sp = plsc.load_gather(cnt_v, [iot * NB + bk])
        ob = plsc.load_gather(off_v, [bk])
        pos = ob + sp + lr
        plsc.store_scatter(sidx_v, [pos], idxv)
        plsc.store_scatter(undo_v, [idxv], pos)
        plsc.store_scatter(skey_v, [pos], bk * S + idxv)
        rv = plsc.load_gather(rn_v, [idxv])
        plsc.store_scatter(srn_v, [pos], rv)
        return 0
    lax.fori_loop(0, STRIDE, p4, 0)

    pltpu.sync_copy(skey_v, skey_hbm.at[w])
    pltpu.sync_copy(undo_v, undo_hbm.at[w])
    pltpu.sync_copy(srn_v, srn_hbm.at[w])

    _pipelined_gather(qkv_hbm.at[w], sidx_v, sqkv_hbm.at[w],
                      (buf, buf2), (sem, sem2))


def _pipelined_gather(table, idx_v, out, bufs, sems):
    nch = S // GR
    cps = [None, None]
    for j in range(nch):
        cps[j % 2] = pltpu.async_copy(
            table.at[idx_v.at[pl.ds(j * GR, GR)]], bufs[j % 2], sems[j % 2])
        if j >= 1:
            cps[(j - 1) % 2].wait()
            pltpu.sync_copy(bufs[(j - 1) % 2],
                            out.at[pl.ds((j - 1) * GR, GR)])
    cps[(nch - 1) % 2].wait()
    pltpu.sync_copy(bufs[(nch - 1) % 2], out.at[pl.ds((nch - 1) * GR, GR)])


def _k2(bkt, rn, qkv):
    mesh = plsc.VectorSubcoreMesh(core_axis_name="c", subcore_axis_name="s",
                                  num_cores=NC)
    f = functools.partial(
        pl.kernel,
        out_type=(
            jax.ShapeDtypeStruct((P, S), jnp.int32),
            jax.ShapeDtypeStruct((P, S), jnp.int32),
            jax.ShapeDtypeStruct((P, S), jnp.float32),
            jax.ShapeDtypeStruct((P, S, 2 * HD), jnp.float32),
        ),
        mesh=mesh,
        scratch_types=[
            pltpu.VMEM((S,), jnp.int32),
            pltpu.VMEM((NB * LANES,), jnp.int32),
            pltpu.VMEM((NB,), jnp.int32),
            pltpu.VMEM((NB,), jnp.int32),
            pltpu.VMEM((S,), jnp.int32),
            pltpu.VMEM((S,), jnp.int32),
            pltpu.VMEM((S,), jnp.int32),
            pltpu.VMEM((S,), jnp.int32),
            pltpu.VMEM((S,), jnp.float32),
            pltpu.VMEM((S,), jnp.float32),
            pltpu.VMEM((GR, 2 * HD), jnp.float32),
            pltpu.VMEM((GR, 2 * HD), jnp.float32),
            pltpu.SemaphoreType.DMA,
            pltpu.SemaphoreType.DMA,
        ],
        compiler_params=pltpu.CompilerParams(needs_layout_passes=False),
    )(_sc_sort_body)
    return f(bkt, rn, qkv)


def _sc_unsort_body(undo_hbm, os_hbm, attn_hbm, undo_v, buf, buf2, sem, sem2):
    w = lax.axis_index("s") * NC + lax.axis_index("c")
    pltpu.sync_copy(undo_hbm.at[w], undo_v)
    _pipelined_gather(os_hbm.at[w], undo_v, attn_hbm.at[w],
                      (buf, buf2), (sem, sem2))


def _k4(undo, osort):
    mesh = plsc.VectorSubcoreMesh(core_axis_name="c", subcore_axis_name="s",
                                  num_cores=NC)
    f = functools.partial(
        pl.kernel,
        out_type=jax.ShapeDtypeStruct((P, S, 2 * HD), jnp.float32),
        mesh=mesh,
        scratch_types=[
            pltpu.VMEM((S,), jnp.int32),
            pltpu.VMEM((GR, 2 * HD), jnp.float32),
            pltpu.VMEM((GR, 2 * HD), jnp.float32),
            pltpu.SemaphoreType.DMA,
            pltpu.SemaphoreType.DMA,
        ],
        compiler_params=pltpu.CompilerParams(needs_layout_passes=False),
    )(_sc_unsort_body)
    return f(undo, osort)


CPB = 8
KBS = CPB * CHUNK
NSB = S // KBS


def _k3_body(sb_ref, pb_ref, nb_ref, ts_ref, tp_ref, tn_ref,
             rs_ref, rp_ref, rn_ref, o_ref):
    sblk = sb_ref[0]
    tsel = ts_ref[0, 0]
    rsel = rs_ref[0, 0]
    eye = (lax.broadcasted_iota(jnp.int32, (CHUNK, CHUNK), 0)
           == lax.broadcasted_iota(jnp.int32, (CHUNK, CHUNK), 1))
    outs = []
    for i in range(CPB):
        lo = i * CHUNK
        q = sblk[lo:lo + CHUNK, :HD]
        qb = (tsel[lo:lo + CHUNK] // S)[:, None]
        qbf = q.astype(jnp.bfloat16)
        if i == 0:
            kvp = pb_ref[0]
            tkp = tp_ref[0, 0, (CPB - 1) * CHUNK:]
            rkp = rp_ref[0, 0, (CPB - 1) * CHUNK:]
        else:
            kvp = sblk[lo - CHUNK:lo, :]
            tkp = tsel[lo - CHUNK:lo]
            rkp = rsel[lo - CHUNK:lo]
        kvs = sblk[lo:lo + CHUNK, :]
        tks = tsel[lo:lo + CHUNK]
        rks = rsel[lo:lo + CHUNK]
        if i == CPB - 1:
            kvn = nb_ref[0]
            tkn = tn_ref[0, 0, :CHUNK]
            rkn = rn_ref[0, 0, :CHUNK]
        else:
            kvn = sblk[lo + CHUNK:lo + 2 * CHUNK, :]
            tkn = tsel[lo + CHUNK:lo + 2 * CHUNK]
            rkn = rsel[lo + CHUNK:lo + 2 * CHUNK]
        ds = []
        for j, (kv, tk, rk) in enumerate(
                ((kvp, tkp, rkp), (kvs, tks, rks), (kvn, tkn, rkn))):
            d = jax.lax.dot_general(
                qbf, kv[:, :HD].astype(jnp.bfloat16),
                (((1,), (1,)), ((), ())),
                preferred_element_type=jnp.float32) * rk[None, :]
            d = jnp.where(qb != (tk // S)[None, :], d - 1e9, d)
            if j == 1:
                d = jnp.where(eye, d - 1e5, d)
            ds.append(d)
        m = jnp.maximum(jnp.max(ds[0], axis=-1, keepdims=True),
                        jnp.maximum(jnp.max(ds[1], axis=-1, keepdims=True),
                                    jnp.max(ds[2], axis=-1, keepdims=True)))
        es = [jnp.exp(d - m) for d in ds]
        tot = (jnp.sum(es[0], axis=-1, keepdims=True)
               + jnp.sum(es[1], axis=-1, keepdims=True)
               + jnp.sum(es[2], axis=-1, keepdims=True))
        o = (jnp.dot(es[0].astype(jnp.bfloat16), kvp[:, HD:].astype(jnp.bfloat16),
                     preferred_element_type=jnp.float32)
             + jnp.dot(es[1].astype(jnp.bfloat16), kvs[:, HD:].astype(jnp.bfloat16),
                       preferred_element_type=jnp.float32)
             + jnp.dot(es[2].astype(jnp.bfloat16), kvn[:, HD:].astype(jnp.bfloat16),
                       preferred_element_type=jnp.float32))
        o = o * (1.0 / tot)
        outs.append(jnp.concatenate([o, jnp.zeros_like(o)], axis=1))
    o_ref[0] = jnp.concatenate(outs, axis=0)


def _k3(sqkv, skey3, srn3):
    def prev(p, c):
        return (p, (c * CPB + NCHUNK - 1) % NCHUNK, 0)

    def nxt(p, c):
        return (p, (c * CPB + CPB) % NCHUNK, 0)

    def tprev(p, c):
        return (p * NSB + (c + NSB - 1) % NSB, 0, 0)

    def tnxt(p, c):
        return (p * NSB + (c + 1) % NSB, 0, 0)

    def tself(p, c):
        return (p * NSB + c, 0, 0)

    return pl.pallas_call(
        _k3_body,
        grid=(P, NSB),
        in_specs=[
            pl.BlockSpec((1, KBS, 2 * HD), lambda p, c: (p, c, 0)),
            pl.BlockSpec((1, CHUNK, 2 * HD), prev),
            pl.BlockSpec((1, CHUNK, 2 * HD), nxt),
            pl.BlockSpec((1, 1, KBS), tself),
            pl.BlockSpec((1, 1, KBS), tprev),
            pl.BlockSpec((1, 1, KBS), tnxt),
            pl.BlockSpec((1, 1, KBS), tself),
            pl.BlockSpec((1, 1, KBS), tprev),
            pl.BlockSpec((1, 1, KBS), tnxt),
        ],
        out_specs=pl.BlockSpec((1, KBS, 2 * HD), lambda p, c: (p, c, 0)),
        out_shape=jax.ShapeDtypeStruct((P, S, 2 * HD), jnp.float32),
    )(sqkv, sqkv, sqkv, skey3, skey3, skey3, srn3, srn3, srn3)


def _k5_body(a_ref, x_ref, wo_ref, s2_ref, b2_ref, x2_ref, yn_ref):
    cat = jnp.concatenate([a_ref[h, :, :HD] for h in range(H)], axis=1)
    out = jnp.dot(cat.astype(jnp.bfloat16), wo_ref[...],
                  preferred_element_type=jnp.float32)
    x2 = x_ref[0] + out
    x2_ref[0] = x2
    mu = jnp.mean(x2, axis=-1, keepdims=True)
    var = jnp.mean(jnp.square(x2 - mu), axis=-1, keepdims=True)
    yn = (x2 - mu) * lax.rsqrt(var + 1e-6) * s2_ref[...] + b2_ref[...]
    yn_ref[0] = yn.astype(jnp.bfloat16)


def _k5(attn, x, wo2, s2, b2):
    bs = 512
    return pl.pallas_call(
        _k5_body,
        grid=(B, S // bs),
        in_specs=[
            pl.BlockSpec((H, bs, 2 * HD), lambda b, s: (b, s, 0)),
            pl.BlockSpec((1, bs, D), lambda b, s: (b, s, 0)),
            pl.BlockSpec((H * HD, D), lambda b, s: (0, 0)),
            pl.BlockSpec((D,), lambda b, s: (0,)),
            pl.BlockSpec((D,), lambda b, s: (0,)),
        ],
        out_specs=[
            pl.BlockSpec((1, bs, D), lambda b, s: (b, s, 0)),
            pl.BlockSpec((1, bs, D), lambda b, s: (b, s, 0)),
        ],
        out_shape=[
            jax.ShapeDtypeStruct((B, S, D), jnp.float32),
            jax.ShapeDtypeStruct((B, S, D), jnp.bfloat16),
        ],
    )(attn, x, wo2, s2, b2)


def _k6_body(yn_ref, w1_ref, b1_ref, w2_ref, x2_ref, b2_ref, out_ref,
             acc_ref):
    n = pl.program_id(1)
    nblocks = pl.num_programs(1)

    @pl.when(n == 0)
    def _():
        acc_ref[...] = jnp.zeros_like(acc_ref)

    hid = jnp.dot(yn_ref[...], w1_ref[...],
                  preferred_element_type=jnp.float32) + b1_ref[...]
    hid = jnp.maximum(hid, 0.0).astype(jnp.bfloat16)
    acc_ref[...] += jnp.dot(hid, w2_ref[...],
                            preferred_element_type=jnp.float32)

    @pl.when(n == nblocks - 1)
    def _():
        out_ref[...] = acc_ref[...] + x2_ref[...] + b2_ref[...]


def _k6(yn2, w1, b1, w2, x22, b2):
    rb = 1024
    nb = 1024
    rows = B * S
    return pl.pallas_call(
        _k6_body,
        grid=(rows // rb, MLP_D // nb),
        in_specs=[
            pl.BlockSpec((rb, D), lambda r, n: (r, 0)),
            pl.BlockSpec((D, nb), lambda r, n: (0, n)),
            pl.BlockSpec((nb,), lambda r, n: (n,)),
            pl.BlockSpec((nb, D), lambda r, n: (n, 0)),
            pl.BlockSpec((rb, D), lambda r, n: (r, 0)),
            pl.BlockSpec((D,), lambda r, n: (0,)),
        ],
        out_specs=pl.BlockSpec((rb, D), lambda r, n: (r, 0)),
        out_shape=jax.ShapeDtypeStruct((rows, D), jnp.float32),
        scratch_shapes=[pltpu.VMEM((rb, D), jnp.float32)],
    )(yn2, w1, b1, w2, x22, b2)


def kernel(inputs, ln1_scale, ln1_bias, Wqk, Wv, Wo, rot, ln2_scale,
           ln2_bias, W1, b1, W2, b2):
    wqk2 = Wqk.reshape(D, H * HD)
    wv2 = Wv.reshape(D, H * HD).astype(jnp.bfloat16)
    wo2 = Wo.reshape(H * HD, D).astype(jnp.bfloat16)
    rotbd = jax.scipy.linalg.block_diag(*[rot[h] for h in range(H)])

    qkv4, bkt3, rn4 = _k1(inputs, ln1_scale, ln1_bias, wqk2, wv2, rotbd)
    qkv = qkv4.reshape(P, S, 2 * HD)
    bkt = bkt3.reshape(P, S)
    rn = rn4.reshape(P, S)

    skey, undo, srn, sqkv = _k2(bkt, rn, qkv)
    skey3 = skey.reshape(P * NSB, 1, KBS)
    srn3 = srn.reshape(P * NSB, 1, KBS)

    osort = _k3(sqkv, skey3, srn3)
    attn = _k4(undo, osort)

    x2, yn = _k5(attn, inputs, wo2, ln2_scale, ln2_bias)

    final = _k6(yn.reshape(B * S, D), W1.astype(jnp.bfloat16), b1,
                W2.astype(jnp.bfloat16), x2.reshape(B * S, D), b2)
    return final.reshape(B, S, D)


# speedup vs baseline: 7.1930x; 1.0755x over previous
import functools

import jax
import jax.numpy as jnp
from jax import lax
from jax.experimental import pallas as pl
from jax.experimental.pallas import tpu as pltpu
from jax.experimental.pallas import tpu_sc as plsc

B = 2
S = 4096
D = 1024
H = 16
HD = 64
MLP_D = 4096
CHUNK = 128
NB = 64
P = B * H
NCHUNK = S // CHUNK

NC = 2
NS = 16
LANES = 16
STRIDE = S // LANES
GR = 256


def _k1_body(x_ref, s1_ref, b1_ref, wqk_ref, wv_ref, rot_ref,
             qkv_ref, bkt_ref, rn_ref):
    x = x_ref[0]
    mu = jnp.mean(x, axis=-1, keepdims=True)
    var = jnp.mean(jnp.square(x - mu), axis=-1, keepdims=True)
    xn = (x - mu) * lax.rsqrt(var + 1e-6) * s1_ref[...] + b1_ref[...]
    qk = jnp.dot(xn, wqk_ref[...], preferred_element_type=jnp.float32)
    v = jnp.dot(xn.astype(jnp.bfloat16), wv_ref[...],
                preferred_element_type=jnp.float32)
    bs = x.shape[0]
    rall = jnp.dot(qk, rot_ref[...], preferred_element_type=jnp.float32)
    aabs = jnp.abs(rall)
    codes = (lax.broadcasted_iota(jnp.int32, (bs, H * NB // 2), 1) % (NB // 2)
             + jnp.where(rall < 0.0, NB // 2, 0))
    for h in range(H):
        qh = qk[:, h * HD:(h + 1) * HD]
        qkv_ref[0, h, :, 0:HD] = qh
        qkv_ref[0, h, :, HD:2 * HD] = v[:, h * HD:(h + 1) * HD]
        ss = jnp.sum(jnp.square(qh), axis=-1, keepdims=True)
        rn_ref[0, :, h:h + 1] = 0.125 / (jnp.sqrt(ss) + 1e-6)
        a = aabs[:, h * (NB // 2):(h + 1) * (NB // 2)]
        m = jnp.max(a, axis=-1, keepdims=True)
        cand = jnp.where(a == m, codes[:, h * (NB // 2):(h + 1) * (NB // 2)],
                         NB)
        bkt_ref[0, :, h:h + 1] = jnp.min(cand, axis=-1, keepdims=True)


def _k1(x, s1, b1, wqk2, wv2, rot):
    bs = 512
    grid = (B, S // bs)
    return pl.pallas_call(
        _k1_body,
        grid=grid,
        in_specs=[
            pl.BlockSpec((1, bs, D), lambda b, s: (b, s, 0)),
            pl.BlockSpec((D,), lambda b, s: (0,)),
            pl.BlockSpec((D,), lambda b, s: (0,)),
            pl.BlockSpec((D, H * HD), lambda b, s: (0, 0)),
            pl.BlockSpec((D, H * HD), lambda b, s: (0, 0)),
            pl.BlockSpec((H * HD, H * NB // 2), lambda b, s: (0, 0)),
        ],
        out_specs=[
            pl.BlockSpec((1, H, bs, 2 * HD), lambda b, s: (b, 0, s, 0)),
            pl.BlockSpec((1, bs, H), lambda b, s: (b, s, 0)),
            pl.BlockSpec((1, bs, H), lambda b, s: (b, s, 0)),
        ],
        out_shape=[
            jax.ShapeDtypeStruct((B, H, S, 2 * HD), jnp.float32),
            jax.ShapeDtypeStruct((B, S, H), jnp.int32),
            jax.ShapeDtypeStruct((B, S, H), jnp.float32),
        ],
    )(x, s1, b1, wqk2, wv2, rot)


def _sc_sort_body(bkt_hbm, rn_hbm, qkv_hbm,
                  skey_hbm, undo_hbm, srn_hbm, sqkv_hbm,
                  bkt_v, cnt_v, tot_v, off_v, lr_v, sidx_v, undo_v, skey_v,
                  rn_v, srn_v, buf, buf2, sem, sem2):
    w = lax.axis_index("s") * NC + lax.axis_index("c")
    iot = lax.iota(jnp.int32, LANES)
    pltpu.sync_copy(bkt_hbm.at[w], bkt_v)
    pltpu.sync_copy(rn_hbm.at[w], rn_v)

    def zero(i, _):
        cnt_v[pl.ds(i * LANES, LANES)] = jnp.zeros((LANES,), jnp.int32)
        return 0
    lax.fori_loop(0, NB * LANES // LANES, zero, 0)

    def p1(t, _):
        idxv = iot * STRIDE + t
        bk = plsc.load_gather(bkt_v, [idxv])
        cidx = iot * NB + bk
        c = plsc.load_gather(cnt_v, [cidx])
        plsc.store_scatter(lr_v, [idxv], c)
        plsc.store_scatter(cnt_v, [cidx], c + 1)
        return 0
    lax.fori_loop(0, STRIDE, p1, 0)

    def p2(bkt, _):
        colidx = iot * NB + bkt
        c = plsc.load_gather(cnt_v, [colidx])
        s = plsc.cumsum(c)
        plsc.store_scatter(cnt_v, [colidx], s - c)
        plsc.store_scatter(tot_v, [iot * 0 + bkt], s, mask=iot == LANES - 1)
        return 0
    lax.fori_loop(0, NB, p2, 0)

    def p3(g, carry):
        tv = tot_v[pl.ds(g * LANES, LANES)]
        s = plsc.cumsum(tv)
        off_v[pl.ds(g * LANES, LANES)] = s - tv + carry
        return carry + jnp.sum(tv)
    lax.fori_loop(0, NB // LANES, p3, jnp.int32(0))

    def p4(t, _):
        idxv = iot * STRIDE + t
        bk = plsc.load_gather(bkt_v, [idxv])
        lr = plsc.load_gather(lr_v, [idxv])
        sp = plsc.load_gather(cnt_v, [iot * NB + bk])
        ob = plsc.load_gather(off_v, [bk])
        pos = ob + sp + lr
        plsc.store_scatter(sidx_v, [pos], idxv)
        plsc.store_scatter(undo_v, [idxv], pos)
        plsc.store_scatter(skey_v, [pos], bk * S + idxv)
        rv = plsc.load_gather(rn_v, [idxv])
        plsc.store_scatter(srn_v, [pos], rv)
        return 0
    lax.fori_loop(0, STRIDE, p4, 0)

    pltpu.sync_copy(skey_v, skey_hbm.at[w])
    pltpu.sync_copy(undo_v, undo_hbm.at[w])
    pltpu.sync_copy(srn_v, srn_hbm.at[w])

    _pipelined_gather(qkv_hbm.at[w], sidx_v, sqkv_hbm.at[w],
                      (buf, buf2), (sem, sem2))


def _pipelined_gather(table, idx_v, out, bufs, sems):
    nch = S // GR
    cps = [None, None]
    for j in range(nch):
        cps[j % 2] = pltpu.async_copy(
            table.at[idx_v.at[pl.ds(j * GR, GR)]], bufs[j % 2], sems[j % 2])
        if j >= 1:
            cps[(j - 1) % 2].wait()
            pltpu.sync_copy(bufs[(j - 1) % 2],
                            out.at[pl.ds((j - 1) * GR, GR)])
    cps[(nch - 1) % 2].wait()
    pltpu.sync_copy(bufs[(nch - 1) % 2], out.at[pl.ds((nch - 1) * GR, GR)])


def _k2(bkt, rn, qkv):
    mesh = plsc.VectorSubcoreMesh(core_axis_name="c", subcore_axis_name="s",
                                  num_cores=NC)
    f = functools.partial(
        pl.kernel,
        out_type=(
            jax.ShapeDtypeStruct((P, S), jnp.int32),
            jax.ShapeDtypeStruct((P, S), jnp.int32),
            jax.ShapeDtypeStruct((P, S), jnp.float32),
            jax.ShapeDtypeStruct((P, S, 2 * HD), jnp.float32),
        ),
        mesh=mesh,
        scratch_types=[
            pltpu.VMEM((S,), jnp.int32),
            pltpu.VMEM((NB * LANES,), jnp.int32),
            pltpu.VMEM((NB,), jnp.int32),
            pltpu.VMEM((NB,), jnp.int32),
            pltpu.VMEM((S,), jnp.int32),
            pltpu.VMEM((S,), jnp.int32),
            pltpu.VMEM((S,), jnp.int32),
            pltpu.VMEM((S,), jnp.int32),
            pltpu.VMEM((S,), jnp.float32),
            pltpu.VMEM((S,), jnp.float32),
            pltpu.VMEM((GR, 2 * HD), jnp.float32),
            pltpu.VMEM((GR, 2 * HD), jnp.float32),
            pltpu.SemaphoreType.DMA,
            pltpu.SemaphoreType.DMA,
        ],
        compiler_params=pltpu.CompilerParams(needs_layout_passes=False),
    )(_sc_sort_body)
    return f(bkt, rn, qkv)


def _sc_unsort_body(undo_hbm, os_hbm, attn_hbm, undo_v, buf, buf2, sem, sem2):
    w = lax.axis_index("s") * NC + lax.axis_index("c")
    pltpu.sync_copy(undo_hbm.at[w], undo_v)
    _pipelined_gather(os_hbm.at[w], undo_v, attn_hbm.at[w],
                      (buf, buf2), (sem, sem2))


def _k4(undo, osort):
    mesh = plsc.VectorSubcoreMesh(core_axis_name="c", subcore_axis_name="s",
                                  num_cores=NC)
    f = functools.partial(
        pl.kernel,
        out_type=jax.ShapeDtypeStruct((P, S, 2 * HD), jnp.float32),
        mesh=mesh,
        scratch_types=[
            pltpu.VMEM((S,), jnp.int32),
            pltpu.VMEM((GR, 2 * HD), jnp.float32),
            pltpu.VMEM((GR, 2 * HD), jnp.float32),
            pltpu.SemaphoreType.DMA,
            pltpu.SemaphoreType.DMA,
        ],
        compiler_params=pltpu.CompilerParams(needs_layout_passes=False),
    )(_sc_unsort_body)
    return f(undo, osort)


CPB = 8
KBS = CPB * CHUNK
NSB = S // KBS


def _k3_body(sb_ref, pb_ref, nb_ref, ts_ref, tp_ref, tn_ref,
             rs_ref, rp_ref, rn_ref, o_ref):
    sblk = sb_ref[0]
    tsel = ts_ref[0, 0]
    rsel = rs_ref[0, 0]
    eye = (lax.broadcasted_iota(jnp.int32, (CHUNK, CHUNK), 0)
           == lax.broadcasted_iota(jnp.int32, (CHUNK, CHUNK), 1))
    outs = []
    for i in range(CPB):
        lo = i * CHUNK
        q = sblk[lo:lo + CHUNK, :HD]
        qb = (tsel[lo:lo + CHUNK] // S)[:, None]
        qbf = q.astype(jnp.bfloat16)
        if i == 0:
            kvp = pb_ref[0]
            tkp = tp_ref[0, 0, (CPB - 1) * CHUNK:]
            rkp = rp_ref[0, 0, (CPB - 1) * CHUNK:]
        else:
            kvp = sblk[lo - CHUNK:lo, :]
            tkp = tsel[lo - CHUNK:lo]
            rkp = rsel[lo - CHUNK:lo]
        kvs = sblk[lo:lo + CHUNK, :]
        tks = tsel[lo:lo + CHUNK]
        rks = rsel[lo:lo + CHUNK]
        if i == CPB - 1:
            kvn = nb_ref[0]
            tkn = tn_ref[0, 0, :CHUNK]
            rkn = rn_ref[0, 0, :CHUNK]
        else:
            kvn = sblk[lo + CHUNK:lo + 2 * CHUNK, :]
            tkn = tsel[lo + CHUNK:lo + 2 * CHUNK]
            rkn = rsel[lo + CHUNK:lo + 2 * CHUNK]
        ds = []
        for j, (kv, tk, rk) in enumerate(
                ((kvp, tkp, rkp), (kvs, tks, rks), (kvn, tkn, rkn))):
            d = jax.lax.dot_general(
                qbf, kv[:, :HD].astype(jnp.bfloat16),
                (((1,), (1,)), ((), ())),
                preferred_element_type=jnp.float32) * rk[None, :]
            d = jnp.where(qb != (tk // S)[None, :], d - 1e9, d)
            if j == 1:
                d = jnp.where(eye, d - 1e5, d)
            ds.append(d)
        m = jnp.maximum(jnp.max(ds[0], axis=-1, keepdims=True),
                        jnp.maximum(jnp.max(ds[1], axis=-1, keepdims=True),
                                    jnp.max(ds[2], axis=-1, keepdims=True)))
        es = [jnp.exp(d - m) for d in ds]
        tot = (jnp.sum(es[0], axis=-1, keepdims=True)
               + jnp.sum(es[1], axis=-1, keepdims=True)
               + jnp.sum(es[2], axis=-1, keepdims=True))
        o = (jnp.dot(es[0].astype(jnp.bfloat16), kvp[:, HD:].astype(jnp.bfloat16),
                     preferred_element_type=jnp.float32)
             + jnp.dot(es[1].astype(jnp.bfloat16), kvs[:, HD:].astype(jnp.bfloat16),
                       preferred_element_type=jnp.float32)
             + jnp.dot(es[2].astype(jnp.bfloat16), kvn[:, HD:].astype(jnp.bfloat16),
                       preferred_element_type=jnp.float32))
        o = o * (1.0 / tot)
        outs.append(jnp.concatenate([o, jnp.zeros_like(o)], axis=1))
    o_ref[0] = jnp.concatenate(outs, axis=0)


def _k3(sqkv, skey3, srn3):
    def prev(p, c):
        return (p, (c * CPB + NCHUNK - 1) % NCHUNK, 0)

    def nxt(p, c):
        return (p, (c * CPB + CPB) % NCHUNK, 0)

    def tprev(p, c):
        return (p * NSB + (c + NSB - 1) % NSB, 0, 0)

    def tnxt(p, c):
        return (p * NSB + (c + 1) % NSB, 0, 0)

    def tself(p, c):
        return (p * NSB + c, 0, 0)

    return pl.pallas_call(
        _k3_body,
        grid=(P, NSB),
        in_specs=[
            pl.BlockSpec((1, KBS, 2 * HD), lambda p, c: (p, c, 0)),
            pl.BlockSpec((1, CHUNK, 2 * HD), prev),
            pl.BlockSpec((1, CHUNK, 2 * HD), nxt),
            pl.BlockSpec((1, 1, KBS), tself),
            pl.BlockSpec((1, 1, KBS), tprev),
            pl.BlockSpec((1, 1, KBS), tnxt),
            pl.BlockSpec((1, 1, KBS), tself),
            pl.BlockSpec((1, 1, KBS), tprev),
            pl.BlockSpec((1, 1, KBS), tnxt),
        ],
        out_specs=pl.BlockSpec((1, KBS, 2 * HD), lambda p, c: (p, c, 0)),
        out_shape=jax.ShapeDtypeStruct((P, S, 2 * HD), jnp.float32),
    )(sqkv, sqkv, sqkv, skey3, skey3, skey3, srn3, srn3, srn3)


def _k5_body(a_ref, x_ref, wo_ref, s2_ref, b2_ref, x2_ref, yn_ref):
    cat = jnp.concatenate([a_ref[h, :, :HD] for h in range(H)], axis=1)
    out = jnp.dot(cat.astype(jnp.bfloat16), wo_ref[...],
                  preferred_element_type=jnp.float32)
    x2 = x_ref[0] + out
    x2_ref[0] = x2
    mu = jnp.mean(x2, axis=-1, keepdims=True)
    var = jnp.mean(jnp.square(x2 - mu), axis=-1, keepdims=True)
    yn = (x2 - mu) * lax.rsqrt(var + 1e-6) * s2_ref[...] + b2_ref[...]
    yn_ref[0] = yn.astype(jnp.bfloat16)


def _k5(attn, x, wo2, s2, b2):
    bs = 512
    return pl.pallas_call(
        _k5_body,
        grid=(B, S // bs),
        in_specs=[
            pl.BlockSpec((H, bs, 2 * HD), lambda b, s: (b, s, 0)),
            pl.BlockSpec((1, bs, D), lambda b, s: (b, s, 0)),
            pl.BlockSpec((H * HD, D), lambda b, s: (0, 0)),
            pl.BlockSpec((D,), lambda b, s: (0,)),
            pl.BlockSpec((D,), lambda b, s: (0,)),
        ],
        out_specs=[
            pl.BlockSpec((1, bs, D), lambda b, s: (b, s, 0)),
            pl.BlockSpec((1, bs, D), lambda b, s: (b, s, 0)),
        ],
        out_shape=[
            jax.ShapeDtypeStruct((B, S, D), jnp.float32),
            jax.ShapeDtypeStruct((B, S, D), jnp.bfloat16),
        ],
    )(attn, x, wo2, s2, b2)


def _k6_body(yn_ref, w1_ref, b1_ref, w2_ref, x2_ref, b2_ref, out_ref,
             acc_ref):
    n = pl.program_id(1)
    nblocks = pl.num_programs(1)

    @pl.when(n == 0)
    def _():
        acc_ref[...] = jnp.zeros_like(acc_ref)

    hid = jnp.dot(yn_ref[...], w1_ref[...],
                  preferred_element_type=jnp.float32) + b1_ref[...]
    hid = jnp.maximum(hid, 0.0).astype(jnp.bfloat16)
    acc_ref[...] += jnp.dot(hid, w2_ref[...],
                            preferred_element_type=jnp.float32)

    @pl.when(n == nblocks - 1)
    def _():
        out_ref[...] = acc_ref[...] + x2_ref[...] + b2_ref[...]


def _k6(yn2, w1, b1, w2, x22, b2):
    rb = 1024
    nb = 1024
    rows = B * S
    return pl.pallas_call(
        _k6_body,
        grid=(rows // rb, MLP_D // nb),
        in_specs=[
            pl.BlockSpec((rb, D), lambda r, n: (r, 0)),
            pl.BlockSpec((D, nb), lambda r, n: (0, n)),
            pl.BlockSpec((nb,), lambda r, n: (n,)),
            pl.BlockSpec((nb, D), lambda r, n: (n, 0)),
            pl.BlockSpec((rb, D), lambda r, n: (r, 0)),
            pl.BlockSpec((D,), lambda r, n: (0,)),
        ],
        out_specs=pl.BlockSpec((rb, D), lambda r, n: (r, 0)),
        out_shape=jax.ShapeDtypeStruct((rows, D), jnp.float32),
        scratch_shapes=[pltpu.VMEM((rb, D), jnp.float32)],
    )(yn2, w1, b1, w2, x22, b2)


def kernel(inputs, ln1_scale, ln1_bias, Wqk, Wv, Wo, rot, ln2_scale,
           ln2_bias, W1, b1, W2, b2):
    wqk2 = Wqk.reshape(D, H * HD)
    wv2 = Wv.reshape(D, H * HD).astype(jnp.bfloat16)
    wo2 = Wo.reshape(H * HD, D).astype(jnp.bfloat16)
    rotbd = jax.scipy.linalg.block_diag(*[rot[h] for h in range(H)])

    qkv4, bkt3, rn4 = _k1(inputs, ln1_scale, ln1_bias, wqk2, wv2, rotbd)
    qkv = qkv4.reshape(P, S, 2 * HD)
    bkt = jnp.transpose(bkt3, (0, 2, 1)).reshape(P, S)
    rn = jnp.transpose(rn4, (0, 2, 1)).reshape(P, S)

    skey, undo, srn, sqkv = _k2(bkt, rn, qkv)
    skey3 = skey.reshape(P * NSB, 1, KBS)
    srn3 = srn.reshape(P * NSB, 1, KBS)

    osort = _k3(sqkv, skey3, srn3)
    attn = _k4(undo, osort)

    x2, yn = _k5(attn, inputs, wo2, ln2_scale, ln2_bias)

    final = _k6(yn.reshape(B * S, D), W1.astype(jnp.bfloat16), b1,
                W2.astype(jnp.bfloat16), x2.reshape(B * S, D), b2)
    return final.reshape(B, S, D)


# speedup vs baseline: 7.4045x; 1.0294x over previous
import functools

import jax
import jax.numpy as jnp
from jax import lax
from jax.experimental import pallas as pl
from jax.experimental.pallas import tpu as pltpu
from jax.experimental.pallas import tpu_sc as plsc

B = 2
S = 4096
D = 1024
H = 16
HD = 64
MLP_D = 4096
CHUNK = 128
NB = 64
P = B * H
NCHUNK = S // CHUNK

NC = 2
NS = 16
LANES = 16
STRIDE = S // LANES
GR = 256


def _k1_body(x_ref, s1_ref, b1_ref, wqk_ref, wv_ref, rot_ref,
             qkv_ref, bkt_ref, rn_ref):
    x = x_ref[0]
    mu = jnp.mean(x, axis=-1, keepdims=True)
    var = jnp.mean(jnp.square(x - mu), axis=-1, keepdims=True)
    xn = (x - mu) * lax.rsqrt(var + 1e-6) * s1_ref[...] + b1_ref[...]
    qk = jnp.dot(xn, wqk_ref[...], preferred_element_type=jnp.float32)
    v = jnp.dot(xn.astype(jnp.bfloat16), wv_ref[...],
                preferred_element_type=jnp.float32)
    bs = x.shape[0]
    rall = jnp.dot(qk, rot_ref[...], preferred_element_type=jnp.float32)
    aabs = jnp.abs(rall)
    codes = (lax.broadcasted_iota(jnp.int32, (bs, H * NB // 2), 1) % (NB // 2)
             + jnp.where(rall < 0.0, NB // 2, 0))
    for h in range(H):
        qh = qk[:, h * HD:(h + 1) * HD]
        qkv_ref[0, h, :, 0:HD] = qh
        qkv_ref[0, h, :, HD:2 * HD] = v[:, h * HD:(h + 1) * HD]
        ss = jnp.sum(jnp.square(qh), axis=-1, keepdims=True)
        rn_ref[0, :, h:h + 1] = 0.125 / (jnp.sqrt(ss) + 1e-6)
        a = aabs[:, h * (NB // 2):(h + 1) * (NB // 2)]
        m = jnp.max(a, axis=-1, keepdims=True)
        cand = jnp.where(a == m, codes[:, h * (NB // 2):(h + 1) * (NB // 2)],
                         NB)
        bkt_ref[0, :, h:h + 1] = jnp.min(cand, axis=-1, keepdims=True)


def _k1(x, s1, b1, wqk2, wv2, rot):
    bs = 512
    grid = (B, S // bs)
    return pl.pallas_call(
        _k1_body,
        grid=grid,
        in_specs=[
            pl.BlockSpec((1, bs, D), lambda b, s: (b, s, 0)),
            pl.BlockSpec((D,), lambda b, s: (0,)),
            pl.BlockSpec((D,), lambda b, s: (0,)),
            pl.BlockSpec((D, H * HD), lambda b, s: (0, 0)),
            pl.BlockSpec((D, H * HD), lambda b, s: (0, 0)),
            pl.BlockSpec((H * HD, H * NB // 2), lambda b, s: (0, 0)),
        ],
        out_specs=[
            pl.BlockSpec((1, H, bs, 2 * HD), lambda b, s: (b, 0, s, 0)),
            pl.BlockSpec((1, bs, H), lambda b, s: (b, s, 0)),
            pl.BlockSpec((1, bs, H), lambda b, s: (b, s, 0)),
        ],
        out_shape=[
            jax.ShapeDtypeStruct((B, H, S, 2 * HD), jnp.float32),
            jax.ShapeDtypeStruct((B, S, H), jnp.int32),
            jax.ShapeDtypeStruct((B, S, H), jnp.float32),
        ],
    )(x, s1, b1, wqk2, wv2, rot)


def _sc_sort_body(bkt_hbm, rn_hbm, qkv_hbm,
                  skey_hbm, undo_hbm, srn_hbm, sqkv_hbm,
                  bkt_v, cnt_v, tot_v, off_v, lr_v, sidx_v, undo_v, skey_v,
                  rn_v, srn_v, buf, buf2, sem, sem2):
    w = lax.axis_index("s") * NC + lax.axis_index("c")
    iot = lax.iota(jnp.int32, LANES)
    pltpu.sync_copy(bkt_hbm.at[w], bkt_v)
    pltpu.sync_copy(rn_hbm.at[w], rn_v)

    def zero(i, _):
        cnt_v[pl.ds(i * LANES, LANES)] = jnp.zeros((LANES,), jnp.int32)
        return 0
    lax.fori_loop(0, NB * LANES // LANES, zero, 0)

    def p1(t, _):
        idxv = iot * STRIDE + t
        bk = plsc.load_gather(bkt_v, [idxv])
        cidx = iot * NB + bk
        c = plsc.load_gather(cnt_v, [cidx])
        plsc.store_scatter(lr_v, [idxv], c)
        plsc.store_scatter(cnt_v, [cidx], c + 1)
        return 0
    lax.fori_loop(0, STRIDE, p1, 0)

    def p2(bkt, _):
        colidx = iot * NB + bkt
        c = plsc.load_gather(cnt_v, [colidx])
        s = plsc.cumsum(c)
        plsc.store_scatter(cnt_v, [colidx], s - c)
        plsc.store_scatter(tot_v, [iot * 0 + bkt], s, mask=iot == LANES - 1)
        return 0
    lax.fori_loop(0, NB, p2, 0)

    def p3(g, carry):
        tv = tot_v[pl.ds(g * LANES, LANES)]
        s = plsc.cumsum(tv)
        off_v[pl.ds(g * LANES, LANES)] = s - tv + carry
        return carry + jnp.sum(tv)
    lax.fori_loop(0, NB // LANES, p3, jnp.int32(0))

    def p4(t, _):
        idxv = iot * STRIDE + t
        bk = plsc.load_gather(bkt_v, [idxv])
        lr = plsc.load_gather(lr_v, [idxv])
        sp = plsc.load_gather(cnt_v, [iot * NB + bk])
        ob = plsc.load_gather(off_v, [bk])
        pos = ob + sp + lr
        plsc.store_scatter(sidx_v, [pos], idxv)
        plsc.store_scatter(undo_v, [idxv], pos)
        plsc.store_scatter(skey_v, [pos], bk * S + idxv)
        rv = plsc.load_gather(rn_v, [idxv])
        plsc.store_scatter(srn_v, [pos], rv)
        return 0
    lax.fori_loop(0, STRIDE, p4, 0)

    pltpu.sync_copy(skey_v, skey_hbm.at[w])
    pltpu.sync_copy(undo_v, undo_hbm.at[w])
    pltpu.sync_copy(srn_v, srn_hbm.at[w])

    _pipelined_gather(qkv_hbm.at[w], sidx_v, sqkv_hbm.at[w],
                      (buf, buf2), (sem, sem2))


def _pipelined_gather(table, idx_v, out, bufs, sems):
    nch = S // GR
    cps = [None, None]
    for j in range(nch):
        cps[j % 2] = pltpu.async_copy(
            table.at[idx_v.at[pl.ds(j * GR, GR)]], bufs[j % 2], sems[j % 2])
        if j >= 1:
            cps[(j - 1) % 2].wait()
            pltpu.sync_copy(bufs[(j - 1) % 2],
                            out.at[pl.ds((j - 1) * GR, GR)])
    cps[(nch - 1) % 2].wait()
    pltpu.sync_copy(bufs[(nch - 1) % 2], out.at[pl.ds((nch - 1) * GR, GR)])


def _k2(bkt, rn, qkv):
    mesh = plsc.VectorSubcoreMesh(core_axis_name="c", subcore_axis_name="s",
                                  num_cores=NC)
    f = functools.partial(
        pl.kernel,
        out_type=(
            jax.ShapeDtypeStruct((P, S), jnp.int32),
            jax.ShapeDtypeStruct((P, S), jnp.int32),
            jax.ShapeDtypeStruct((P, S), jnp.float32),
            jax.ShapeDtypeStruct((P, S, 2 * HD), jnp.float32),
        ),
        mesh=mesh,
        scratch_types=[
            pltpu.VMEM((S,), jnp.int32),
            pltpu.VMEM((NB * LANES,), jnp.int32),
            pltpu.VMEM((NB,), jnp.int32),
            pltpu.VMEM((NB,), jnp.int32),
            pltpu.VMEM((S,), jnp.int32),
            pltpu.VMEM((S,), jnp.int32),
            pltpu.VMEM((S,), jnp.int32),
            pltpu.VMEM((S,), jnp.int32),
            pltpu.VMEM((S,), jnp.float32),
            pltpu.VMEM((S,), jnp.float32),
            pltpu.VMEM((GR, 2 * HD), jnp.float32),
            pltpu.VMEM((GR, 2 * HD), jnp.float32),
            pltpu.SemaphoreType.DMA,
            pltpu.SemaphoreType.DMA,
        ],
        compiler_params=pltpu.CompilerParams(needs_layout_passes=False),
    )(_sc_sort_body)
    return f(bkt, rn, qkv)


def _sc_unsort_body(undo_hbm, os_hbm, attn_hbm, undo_v, buf, buf2, sem, sem2):
    w = lax.axis_index("s") * NC + lax.axis_index("c")
    pltpu.sync_copy(undo_hbm.at[w], undo_v)
    _pipelined_gather(os_hbm.at[w], undo_v, attn_hbm.at[w],
                      (buf, buf2), (sem, sem2))


def _k4(undo, osort):
    mesh = plsc.VectorSubcoreMesh(core_axis_name="c", subcore_axis_name="s",
                                  num_cores=NC)
    f = functools.partial(
        pl.kernel,
        out_type=jax.ShapeDtypeStruct((P, S, 2 * HD), jnp.float32),
        mesh=mesh,
        scratch_types=[
            pltpu.VMEM((S,), jnp.int32),
            pltpu.VMEM((GR, 2 * HD), jnp.float32),
            pltpu.VMEM((GR, 2 * HD), jnp.float32),
            pltpu.SemaphoreType.DMA,
            pltpu.SemaphoreType.DMA,
        ],
        compiler_params=pltpu.CompilerParams(needs_layout_passes=False),
    )(_sc_unsort_body)
    return f(undo, osort)


CPB = 16
KBS = CPB * CHUNK
NSB = S // KBS


def _k3_body(sb_ref, pb_ref, nb_ref, ts_ref, tp_ref, tn_ref,
             rs_ref, rp_ref, rn_ref, o_ref):
    sblk = sb_ref[0]
    tsel = ts_ref[0, 0]
    rsel = rs_ref[0, 0]
    outs = []
    for i in range(CPB):
        lo = i * CHUNK
        qbf = sblk[lo:lo + CHUNK, :HD].astype(jnp.bfloat16)
        qb = (tsel[lo:lo + CHUNK] // S)[:, None]
        if i == 0:
            pieces = [
                (pb_ref[0], tp_ref[0, 0, (CPB - 1) * CHUNK:],
                 rp_ref[0, 0, (CPB - 1) * CHUNK:], None),
                (sblk[0:2 * CHUNK, :], tsel[0:2 * CHUNK],
                 rsel[0:2 * CHUNK], 0),
            ]
        elif i == CPB - 1:
            pieces = [
                (sblk[lo - CHUNK:lo + CHUNK, :], tsel[lo - CHUNK:lo + CHUNK],
                 rsel[lo - CHUNK:lo + CHUNK], CHUNK),
                (nb_ref[0], tn_ref[0, 0, :CHUNK], rn_ref[0, 0, :CHUNK], None),
            ]
        else:
            pieces = [
                (sblk[lo - CHUNK:lo + 2 * CHUNK, :],
                 tsel[lo - CHUNK:lo + 2 * CHUNK],
                 rsel[lo - CHUNK:lo + 2 * CHUNK], CHUNK),
            ]
        ds, vs = [], []
        for kv, tk, rk, doff in pieces:
            wdt = kv.shape[0]
            d = jax.lax.dot_general(
                qbf, kv[:, :HD].astype(jnp.bfloat16),
                (((1,), (1,)), ((), ())),
                preferred_element_type=jnp.float32) * rk[None, :]
            d = jnp.where(qb != (tk // S)[None, :], d - 1e9, d)
            if doff is not None:
                eye = (lax.broadcasted_iota(jnp.int32, (CHUNK, wdt), 1)
                       == lax.broadcasted_iota(jnp.int32, (CHUNK, wdt), 0)
                       + doff)
                d = jnp.where(eye, d - 1e5, d)
            ds.append(d)
            vs.append(kv[:, HD:].astype(jnp.bfloat16))
        if len(ds) == 1:
            m = jnp.max(ds[0], axis=-1, keepdims=True)
            e = jnp.exp(ds[0] - m)
            tot = jnp.sum(e, axis=-1, keepdims=True)
            o = jnp.dot(e.astype(jnp.bfloat16), vs[0],
                        preferred_element_type=jnp.float32)
        else:
            m = jnp.maximum(jnp.max(ds[0], axis=-1, keepdims=True),
                            jnp.max(ds[1], axis=-1, keepdims=True))
            e0 = jnp.exp(ds[0] - m)
            e1 = jnp.exp(ds[1] - m)
            tot = (jnp.sum(e0, axis=-1, keepdims=True)
                   + jnp.sum(e1, axis=-1, keepdims=True))
            o = (jnp.dot(e0.astype(jnp.bfloat16), vs[0],
                         preferred_element_type=jnp.float32)
                 + jnp.dot(e1.astype(jnp.bfloat16), vs[1],
                           preferred_element_type=jnp.float32))
        o = o * (1.0 / tot)
        outs.append(jnp.concatenate([o, jnp.zeros_like(o)], axis=1))
    o_ref[0] = jnp.concatenate(outs, axis=0)


def _k3(sqkv, skey3, srn3):
    def prev(p, c):
        return (p, (c * CPB + NCHUNK - 1) % NCHUNK, 0)

    def nxt(p, c):
        return (p, (c * CPB + CPB) % NCHUNK, 0)

    def tprev(p, c):
        return (p * NSB + (c + NSB - 1) % NSB, 0, 0)

    def tnxt(p, c):
        return (p * NSB + (c + 1) % NSB, 0, 0)

    def tself(p, c):
        return (p * NSB + c, 0, 0)

    return pl.pallas_call(
        _k3_body,
        grid=(P, NSB),
        in_specs=[
            pl.BlockSpec((1, KBS, 2 * HD), lambda p, c: (p, c, 0)),
            pl.BlockSpec((1, CHUNK, 2 * HD), prev),
            pl.BlockSpec((1, CHUNK, 2 * HD), nxt),
            pl.BlockSpec((1, 1, KBS), tself),
            pl.BlockSpec((1, 1, KBS), tprev),
            pl.BlockSpec((1, 1, KBS), tnxt),
            pl.BlockSpec((1, 1, KBS), tself),
            pl.BlockSpec((1, 1, KBS), tprev),
            pl.BlockSpec((1, 1, KBS), tnxt),
        ],
        out_specs=pl.BlockSpec((1, KBS, 2 * HD), lambda p, c: (p, c, 0)),
        out_shape=jax.ShapeDtypeStruct((P, S, 2 * HD), jnp.float32),
    )(sqkv, sqkv, sqkv, skey3, skey3, skey3, srn3, srn3, srn3)


def _k5_body(a_ref, x_ref, wo_ref, s2_ref, b2_ref, x2_ref, yn_ref):
    cat = jnp.concatenate([a_ref[h, :, :HD] for h in range(H)], axis=1)
    out = jnp.dot(cat.astype(jnp.bfloat16), wo_ref[...],
                  preferred_element_type=jnp.float32)
    x2 = x_ref[0] + out
    x2_ref[0] = x2
    mu = jnp.mean(x2, axis=-1, keepdims=True)
    var = jnp.mean(jnp.square(x2 - mu), axis=-1, keepdims=True)
    yn = (x2 - mu) * lax.rsqrt(var + 1e-6) * s2_ref[...] + b2_ref[...]
    yn_ref[0] = yn.astype(jnp.bfloat16)


def _k5(attn, x, wo2, s2, b2):
    bs = 512
    return pl.pallas_call(
        _k5_body,
        grid=(B, S // bs),
        in_specs=[
            pl.BlockSpec((H, bs, 2 * HD), lambda b, s: (b, s, 0)),
            pl.BlockSpec((1, bs, D), lambda b, s: (b, s, 0)),
            pl.BlockSpec((H * HD, D), lambda b, s: (0, 0)),
            pl.BlockSpec((D,), lambda b, s: (0,)),
            pl.BlockSpec((D,), lambda b, s: (0,)),
        ],
        out_specs=[
            pl.BlockSpec((1, bs, D), lambda b, s: (b, s, 0)),
            pl.BlockSpec((1, bs, D), lambda b, s: (b, s, 0)),
        ],
        out_shape=[
            jax.ShapeDtypeStruct((B, S, D), jnp.float32),
            jax.ShapeDtypeStruct((B, S, D), jnp.bfloat16),
        ],
    )(attn, x, wo2, s2, b2)


def _k6_body(yn_ref, w1_ref, b1_ref, w2_ref, x2_ref, b2_ref, out_ref,
             acc_ref):
    n = pl.program_id(1)
    nblocks = pl.num_programs(1)

    @pl.when(n == 0)
    def _():
        acc_ref[...] = jnp.zeros_like(acc_ref)

    hid = jnp.dot(yn_ref[...], w1_ref[...],
                  preferred_element_type=jnp.float32) + b1_ref[...]
    hid = jnp.maximum(hid, 0.0).astype(jnp.bfloat16)
    acc_ref[...] += jnp.dot(hid, w2_ref[...],
                            preferred_element_type=jnp.float32)

    @pl.when(n == nblocks - 1)
    def _():
        out_ref[...] = acc_ref[...] + x2_ref[...] + b2_ref[...]


def _k6(yn2, w1, b1, w2, x22, b2):
    rb = 1024
    nb = 1024
    rows = B * S
    return pl.pallas_call(
        _k6_body,
        grid=(rows // rb, MLP_D // nb),
        in_specs=[
            pl.BlockSpec((rb, D), lambda r, n: (r, 0)),
            pl.BlockSpec((D, nb), lambda r, n: (0, n)),
            pl.BlockSpec((nb,), lambda r, n: (n,)),
            pl.BlockSpec((nb, D), lambda r, n: (n, 0)),
            pl.BlockSpec((rb, D), lambda r, n: (r, 0)),
            pl.BlockSpec((D,), lambda r, n: (0,)),
        ],
        out_specs=pl.BlockSpec((rb, D), lambda r, n: (r, 0)),
        out_shape=jax.ShapeDtypeStruct((rows, D), jnp.float32),
        scratch_shapes=[pltpu.VMEM((rb, D), jnp.float32)],
    )(yn2, w1, b1, w2, x22, b2)


def kernel(inputs, ln1_scale, ln1_bias, Wqk, Wv, Wo, rot, ln2_scale,
           ln2_bias, W1, b1, W2, b2):
    wqk2 = Wqk.reshape(D, H * HD)
    wv2 = Wv.reshape(D, H * HD).astype(jnp.bfloat16)
    wo2 = Wo.reshape(H * HD, D).astype(jnp.bfloat16)
    rotbd = jax.scipy.linalg.block_diag(*[rot[h] for h in range(H)])

    qkv4, bkt3, rn4 = _k1(inputs, ln1_scale, ln1_bias, wqk2, wv2, rotbd)
    qkv = qkv4.reshape(P, S, 2 * HD)
    bkt = jnp.transpose(bkt3, (0, 2, 1)).reshape(P, S)
    rn = jnp.transpose(rn4, (0, 2, 1)).reshape(P, S)

    skey, undo, srn, sqkv = _k2(bkt, rn, qkv)
    skey3 = skey.reshape(P * NSB, 1, KBS)
    srn3 = srn.reshape(P * NSB, 1, KBS)

    osort = _k3(sqkv, skey3, srn3)
    attn = _k4(undo, osort)

    x2, yn = _k5(attn, inputs, wo2, ln2_scale, ln2_bias)

    final = _k6(yn.reshape(B * S, D), W1.astype(jnp.bfloat16), b1,
                W2.astype(jnp.bfloat16), x2.reshape(B * S, D), b2)
    return final.reshape(B, S, D)


# speedup vs baseline: 7.5794x; 1.0236x over previous
import functools

import jax
import jax.numpy as jnp
from jax import lax
from jax.experimental import pallas as pl
from jax.experimental.pallas import tpu as pltpu
from jax.experimental.pallas import tpu_sc as plsc

B = 2
S = 4096
D = 1024
H = 16
HD = 64
MLP_D = 4096
CHUNK = 128
NB = 64
P = B * H
NCHUNK = S // CHUNK

NC = 2
NS = 16
LANES = 16
STRIDE = S // LANES
GR = 256


def _k1_body(x_ref, s1_ref, b1_ref, wqk_ref, wv_ref, rot_ref,
             qkv_ref, bkt_ref, rn_ref):
    x = x_ref[0]
    mu = jnp.mean(x, axis=-1, keepdims=True)
    var = jnp.mean(jnp.square(x - mu), axis=-1, keepdims=True)
    xn = (x - mu) * lax.rsqrt(var + 1e-6) * s1_ref[...] + b1_ref[...]
    qk = jnp.dot(xn, wqk_ref[...], preferred_element_type=jnp.float32)
    v = jnp.dot(xn.astype(jnp.bfloat16), wv_ref[...],
                preferred_element_type=jnp.float32)
    bs = x.shape[0]
    rall = jnp.dot(qk, rot_ref[...], preferred_element_type=jnp.float32)
    aabs = jnp.abs(rall)
    codes = (lax.broadcasted_iota(jnp.int32, (bs, H * NB // 2), 1) % (NB // 2)
             + jnp.where(rall < 0.0, NB // 2, 0))
    for h in range(H):
        qh = qk[:, h * HD:(h + 1) * HD]
        qkv_ref[0, h, :, 0:HD] = qh
        qkv_ref[0, h, :, HD:2 * HD] = v[:, h * HD:(h + 1) * HD]
        ss = jnp.sum(jnp.square(qh), axis=-1, keepdims=True)
        rn_ref[0, :, h:h + 1] = 0.125 / (jnp.sqrt(ss) + 1e-6)
        a = aabs[:, h * (NB // 2):(h + 1) * (NB // 2)]
        m = jnp.max(a, axis=-1, keepdims=True)
        cand = jnp.where(a == m, codes[:, h * (NB // 2):(h + 1) * (NB // 2)],
                         NB)
        bkt_ref[0, :, h:h + 1] = jnp.min(cand, axis=-1, keepdims=True)


def _k1(x, s1, b1, wqk2, wv2, rot):
    bs = 512
    nb = x.shape[0]
    grid = (nb, S // bs)
    return pl.pallas_call(
        _k1_body,
        grid=grid,
        in_specs=[
            pl.BlockSpec((1, bs, D), lambda b, s: (b, s, 0)),
            pl.BlockSpec((D,), lambda b, s: (0,)),
            pl.BlockSpec((D,), lambda b, s: (0,)),
            pl.BlockSpec((D, H * HD), lambda b, s: (0, 0)),
            pl.BlockSpec((D, H * HD), lambda b, s: (0, 0)),
            pl.BlockSpec((H * HD, H * NB // 2), lambda b, s: (0, 0)),
        ],
        out_specs=[
            pl.BlockSpec((1, H, bs, 2 * HD), lambda b, s: (b, 0, s, 0)),
            pl.BlockSpec((1, bs, H), lambda b, s: (b, s, 0)),
            pl.BlockSpec((1, bs, H), lambda b, s: (b, s, 0)),
        ],
        out_shape=[
            jax.ShapeDtypeStruct((nb, H, S, 2 * HD), jnp.float32),
            jax.ShapeDtypeStruct((nb, S, H), jnp.int32),
            jax.ShapeDtypeStruct((nb, S, H), jnp.float32),
        ],
    )(x, s1, b1, wqk2, wv2, rot)


def _sc_sort_body(bkt_hbm, rn_hbm, qkv_hbm,
                  skey_hbm, undo_hbm, srn_hbm, sqkv_hbm,
                  bkt_v, cnt_v, tot_v, off_v, lr_v, sidx_v, undo_v, skey_v,
                  rn_v, srn_v, buf, buf2, sem, sem2):
    w = lax.axis_index("s")
    half = lax.axis_index("c")
    iot = lax.iota(jnp.int32, LANES)
    pltpu.sync_copy(bkt_hbm.at[w], bkt_v)
    pltpu.sync_copy(rn_hbm.at[w], rn_v)

    def zero(i, _):
        cnt_v[pl.ds(i * LANES, LANES)] = jnp.zeros((LANES,), jnp.int32)
        return 0
    lax.fori_loop(0, NB * LANES // LANES, zero, 0)

    def p1(t, _):
        idxv = iot * STRIDE + t
        bk = plsc.load_gather(bkt_v, [idxv])
        cidx = iot * NB + bk
        c = plsc.load_gather(cnt_v, [cidx])
        plsc.store_scatter(lr_v, [idxv], c)
        plsc.store_scatter(cnt_v, [cidx], c + 1)
        return 0
    lax.fori_loop(0, STRIDE, p1, 0)

    def p2(bkt, _):
        colidx = iot * NB + bkt
        c = plsc.load_gather(cnt_v, [colidx])
        s = plsc.cumsum(c)
        plsc.store_scatter(cnt_v, [colidx], s - c)
        plsc.store_scatter(tot_v, [iot * 0 + bkt], s, mask=iot == LANES - 1)
        return 0
    lax.fori_loop(0, NB, p2, 0)

    def p3(g, carry):
        tv = tot_v[pl.ds(g * LANES, LANES)]
        s = plsc.cumsum(tv)
        off_v[pl.ds(g * LANES, LANES)] = s - tv + carry
        return carry + jnp.sum(tv)
    lax.fori_loop(0, NB // LANES, p3, jnp.int32(0))

    def p4(t, _):
        idxv = iot * STRIDE + t
        bk = plsc.load_gather(bkt_v, [idxv])
        lr = plsc.load_gather(lr_v, [idxv])
        sp = plsc.load_gather(cnt_v, [iot * NB + bk])
        ob = plsc.load_gather(off_v, [bk])
        pos = ob + sp + lr
        plsc.store_scatter(sidx_v, [pos], idxv)
        plsc.store_scatter(undo_v, [idxv], pos)
        plsc.store_scatter(skey_v, [pos], bk * S + idxv)
        rv = plsc.load_gather(rn_v, [idxv])
        plsc.store_scatter(srn_v, [pos], rv)
        return 0
    lax.fori_loop(0, STRIDE, p4, 0)

    @pl.when(half == 0)
    def _():
        pltpu.sync_copy(skey_v, skey_hbm.at[w])
        pltpu.sync_copy(undo_v, undo_hbm.at[w])
        pltpu.sync_copy(srn_v, srn_hbm.at[w])

    _pipelined_gather(qkv_hbm.at[w], sidx_v, sqkv_hbm.at[w], half,
                      (buf, buf2), (sem, sem2))


def _pipelined_gather(table, idx_v, out, half, bufs, sems):
    nch = S // GR // 2
    base = half * nch
    cps = [None, None]
    for j in range(nch):
        cps[j % 2] = pltpu.async_copy(
            table.at[idx_v.at[pl.ds((base + j) * GR, GR)]],
            bufs[j % 2], sems[j % 2])
        if j >= 1:
            cps[(j - 1) % 2].wait()
            pltpu.sync_copy(bufs[(j - 1) % 2],
                            out.at[pl.ds((base + j - 1) * GR, GR)])
    cps[(nch - 1) % 2].wait()
    pltpu.sync_copy(bufs[(nch - 1) % 2],
                    out.at[pl.ds((base + nch - 1) * GR, GR)])


def _k2(bkt, rn, qkv):
    mesh = plsc.VectorSubcoreMesh(core_axis_name="c", subcore_axis_name="s",
                                  num_cores=NC)
    f = functools.partial(
        pl.kernel,
        out_type=(
            jax.ShapeDtypeStruct((H, S), jnp.int32),
            jax.ShapeDtypeStruct((H, S), jnp.int32),
            jax.ShapeDtypeStruct((H, S), jnp.float32),
            jax.ShapeDtypeStruct((H, S, 2 * HD), jnp.float32),
        ),
        mesh=mesh,
        scratch_types=[
            pltpu.VMEM((S,), jnp.int32),
            pltpu.VMEM((NB * LANES,), jnp.int32),
            pltpu.VMEM((NB,), jnp.int32),
            pltpu.VMEM((NB,), jnp.int32),
            pltpu.VMEM((S,), jnp.int32),
            pltpu.VMEM((S,), jnp.int32),
            pltpu.VMEM((S,), jnp.int32),
            pltpu.VMEM((S,), jnp.int32),
            pltpu.VMEM((S,), jnp.float32),
            pltpu.VMEM((S,), jnp.float32),
            pltpu.VMEM((GR, 2 * HD), jnp.float32),
            pltpu.VMEM((GR, 2 * HD), jnp.float32),
            pltpu.SemaphoreType.DMA,
            pltpu.SemaphoreType.DMA,
        ],
        compiler_params=pltpu.CompilerParams(needs_layout_passes=False),
    )(_sc_sort_body)
    return f(bkt, rn, qkv)


def _sc_unsort_body(undo_hbm, os_hbm, attn_hbm, undo_v, buf, buf2, sem, sem2):
    w = lax.axis_index("s")
    half = lax.axis_index("c")
    pltpu.sync_copy(undo_hbm.at[w], undo_v)
    _pipelined_gather(os_hbm.at[w], undo_v, attn_hbm.at[w], half,
                      (buf, buf2), (sem, sem2))


def _k4(undo, osort):
    mesh = plsc.VectorSubcoreMesh(core_axis_name="c", subcore_axis_name="s",
                                  num_cores=NC)
    f = functools.partial(
        pl.kernel,
        out_type=jax.ShapeDtypeStruct((H, S, 2 * HD), jnp.float32),
        mesh=mesh,
        scratch_types=[
            pltpu.VMEM((S,), jnp.int32),
            pltpu.VMEM((GR, 2 * HD), jnp.float32),
            pltpu.VMEM((GR, 2 * HD), jnp.float32),
            pltpu.SemaphoreType.DMA,
            pltpu.SemaphoreType.DMA,
        ],
        compiler_params=pltpu.CompilerParams(needs_layout_passes=False),
    )(_sc_unsort_body)
    return f(undo, osort)


CPB = 16
KBS = CPB * CHUNK
NSB = S // KBS


def _k3_body(sb_ref, pb_ref, nb_ref, ts_ref, tp_ref, tn_ref,
             rs_ref, rp_ref, rn_ref, o_ref):
    sblk = sb_ref[0]
    tsel = ts_ref[0, 0]
    rsel = rs_ref[0, 0]
    outs = []
    for i in range(CPB):
        lo = i * CHUNK
        qbf = sblk[lo:lo + CHUNK, :HD].astype(jnp.bfloat16)
        qb = (tsel[lo:lo + CHUNK] // S)[:, None]
        if i == 0:
            pieces = [
                (pb_ref[0], tp_ref[0, 0, (CPB - 1) * CHUNK:],
                 rp_ref[0, 0, (CPB - 1) * CHUNK:], None),
                (sblk[0:2 * CHUNK, :], tsel[0:2 * CHUNK],
                 rsel[0:2 * CHUNK], 0),
            ]
        elif i == CPB - 1:
            pieces = [
                (sblk[lo - CHUNK:lo + CHUNK, :], tsel[lo - CHUNK:lo + CHUNK],
                 rsel[lo - CHUNK:lo + CHUNK], CHUNK),
                (nb_ref[0], tn_ref[0, 0, :CHUNK], rn_ref[0, 0, :CHUNK], None),
            ]
        else:
            pieces = [
                (sblk[lo - CHUNK:lo + 2 * CHUNK, :],
                 tsel[lo - CHUNK:lo + 2 * CHUNK],
                 rsel[lo - CHUNK:lo + 2 * CHUNK], CHUNK),
            ]
        ds, vs = [], []
        for kv, tk, rk, doff in pieces:
            wdt = kv.shape[0]
            d = jax.lax.dot_general(
                qbf, kv[:, :HD].astype(jnp.bfloat16),
                (((1,), (1,)), ((), ())),
                preferred_element_type=jnp.float32) * rk[None, :]
            d = jnp.where(qb != (tk // S)[None, :], d - 1e9, d)
            if doff is not None:
                eye = (lax.broadcasted_iota(jnp.int32, (CHUNK, wdt), 1)
                       == lax.broadcasted_iota(jnp.int32, (CHUNK, wdt), 0)
                       + doff)
                d = jnp.where(eye, d - 1e5, d)
            ds.append(d)
            vs.append(kv[:, HD:].astype(jnp.bfloat16))
        if len(ds) == 1:
            m = jnp.max(ds[0], axis=-1, keepdims=True)
            e = jnp.exp(ds[0] - m)
            tot = jnp.sum(e, axis=-1, keepdims=True)
            o = jnp.dot(e.astype(jnp.bfloat16), vs[0],
                        preferred_element_type=jnp.float32)
        else:
            m = jnp.maximum(jnp.max(ds[0], axis=-1, keepdims=True),
                            jnp.max(ds[1], axis=-1, keepdims=True))
            e0 = jnp.exp(ds[0] - m)
            e1 = jnp.exp(ds[1] - m)
            tot = (jnp.sum(e0, axis=-1, keepdims=True)
                   + jnp.sum(e1, axis=-1, keepdims=True))
            o = (jnp.dot(e0.astype(jnp.bfloat16), vs[0],
                         preferred_element_type=jnp.float32)
                 + jnp.dot(e1.astype(jnp.bfloat16), vs[1],
                           preferred_element_type=jnp.float32))
        o = o * (1.0 / tot)
        outs.append(jnp.concatenate([o, jnp.zeros_like(o)], axis=1))
    o_ref[0] = jnp.concatenate(outs, axis=0)


def _k3(sqkv, skey3, srn3):
    def prev(p, c):
        return (p, (c * CPB + NCHUNK - 1) % NCHUNK, 0)

    def nxt(p, c):
        return (p, (c * CPB + CPB) % NCHUNK, 0)

    def tprev(p, c):
        return (p * NSB + (c + NSB - 1) % NSB, 0, 0)

    def tnxt(p, c):
        return (p * NSB + (c + 1) % NSB, 0, 0)

    def tself(p, c):
        return (p * NSB + c, 0, 0)

    return pl.pallas_call(
        _k3_body,
        grid=(sqkv.shape[0], NSB),
        in_specs=[
            pl.BlockSpec((1, KBS, 2 * HD), lambda p, c: (p, c, 0)),
            pl.BlockSpec((1, CHUNK, 2 * HD), prev),
            pl.BlockSpec((1, CHUNK, 2 * HD), nxt),
            pl.BlockSpec((1, 1, KBS), tself),
            pl.BlockSpec((1, 1, KBS), tprev),
            pl.BlockSpec((1, 1, KBS), tnxt),
            pl.BlockSpec((1, 1, KBS), tself),
            pl.BlockSpec((1, 1, KBS), tprev),
            pl.BlockSpec((1, 1, KBS), tnxt),
        ],
        out_specs=pl.BlockSpec((1, KBS, 2 * HD), lambda p, c: (p, c, 0)),
        out_shape=jax.ShapeDtypeStruct((sqkv.shape[0], S, 2 * HD),
                                       jnp.float32),
    )(sqkv, sqkv, sqkv, skey3, skey3, skey3, srn3, srn3, srn3)


def _k5_body(a_ref, x_ref, wo_ref, s2_ref, b2_ref, x2_ref, yn_ref):
    cat = jnp.concatenate([a_ref[h, :, :HD] for h in range(H)], axis=1)
    out = jnp.dot(cat.astype(jnp.bfloat16), wo_ref[...],
                  preferred_element_type=jnp.float32)
    x2 = x_ref[0] + out
    x2_ref[0] = x2
    mu = jnp.mean(x2, axis=-1, keepdims=True)
    var = jnp.mean(jnp.square(x2 - mu), axis=-1, keepdims=True)
    yn = (x2 - mu) * lax.rsqrt(var + 1e-6) * s2_ref[...] + b2_ref[...]
    yn_ref[0] = yn.astype(jnp.bfloat16)


def _k5(attn, x, wo2, s2, b2):
    bs = 512
    nb = x.shape[0]
    return pl.pallas_call(
        _k5_body,
        grid=(nb, S // bs),
        in_specs=[
            pl.BlockSpec((H, bs, 2 * HD), lambda b, s: (b, s, 0)),
            pl.BlockSpec((1, bs, D), lambda b, s: (b, s, 0)),
            pl.BlockSpec((H * HD, D), lambda b, s: (0, 0)),
            pl.BlockSpec((D,), lambda b, s: (0,)),
            pl.BlockSpec((D,), lambda b, s: (0,)),
        ],
        out_specs=[
            pl.BlockSpec((1, bs, D), lambda b, s: (b, s, 0)),
            pl.BlockSpec((1, bs, D), lambda b, s: (b, s, 0)),
        ],
        out_shape=[
            jax.ShapeDtypeStruct((nb, S, D), jnp.float32),
            jax.ShapeDtypeStruct((nb, S, D), jnp.bfloat16),
        ],
    )(attn, x, wo2, s2, b2)


def _k6_body(yn_ref, w1_ref, b1_ref, w2_ref, x2_ref, b2_ref, out_ref,
             acc_ref):
    n = pl.program_id(1)
    nblocks = pl.num_programs(1)

    @pl.when(n == 0)
    def _():
        acc_ref[...] = jnp.zeros_like(acc_ref)

    hid = jnp.dot(yn_ref[...], w1_ref[...],
                  preferred_element_type=jnp.float32) + b1_ref[...]
    hid = jnp.maximum(hid, 0.0).astype(jnp.bfloat16)
    acc_ref[...] += jnp.dot(hid, w2_ref[...],
                            preferred_element_type=jnp.float32)

    @pl.when(n == nblocks - 1)
    def _():
        out_ref[...] = acc_ref[...] + x2_ref[...] + b2_ref[...]


def _k6(yn2, w1, b1, w2, x22, b2):
    rb = 1024
    nb = 1024
    rows = B * S
    return pl.pallas_call(
        _k6_body,
        grid=(rows // rb, MLP_D // nb),
        in_specs=[
            pl.BlockSpec((rb, D), lambda r, n: (r, 0)),
            pl.BlockSpec((D, nb), lambda r, n: (0, n)),
            pl.BlockSpec((nb,), lambda r, n: (n,)),
            pl.BlockSpec((nb, D), lambda r, n: (n, 0)),
            pl.BlockSpec((rb, D), lambda r, n: (r, 0)),
            pl.BlockSpec((D,), lambda r, n: (0,)),
        ],
        out_specs=pl.BlockSpec((rb, D), lambda r, n: (r, 0)),
        out_shape=jax.ShapeDtypeStruct((rows, D), jnp.float32),
        scratch_shapes=[pltpu.VMEM((rb, D), jnp.float32)],
    )(yn2, w1, b1, w2, x22, b2)


def kernel(inputs, ln1_scale, ln1_bias, Wqk, Wv, Wo, rot, ln2_scale,
           ln2_bias, W1, b1, W2, b2):
    wqk2 = Wqk.reshape(D, H * HD)
    wv2 = Wv.reshape(D, H * HD).astype(jnp.bfloat16)
    wo2 = Wo.reshape(H * HD, D).astype(jnp.bfloat16)
    rotbd = jax.scipy.linalg.block_diag(*[rot[h] for h in range(H)])

    x2s, yns = [], []
    for b in range(B):
        xb = lax.slice_in_dim(inputs, b, b + 1, axis=0)
        qkv4, bkt3, rn4 = _k1(xb, ln1_scale, ln1_bias, wqk2, wv2, rotbd)
        qkv = qkv4.reshape(H, S, 2 * HD)
        bkt = jnp.transpose(bkt3, (0, 2, 1)).reshape(H, S)
        rn = jnp.transpose(rn4, (0, 2, 1)).reshape(H, S)

        skey, undo, srn, sqkv = _k2(bkt, rn, qkv)
        skey3 = skey.reshape(H * NSB, 1, KBS)
        srn3 = srn.reshape(H * NSB, 1, KBS)

        osort = _k3(sqkv, skey3, srn3)
        attn = _k4(undo, osort)

        x2b, ynb = _k5(attn, xb, wo2, ln2_scale, ln2_bias)
        x2s.append(x2b)
        yns.append(ynb)

    x2 = jnp.concatenate(x2s, axis=0)
    yn = jnp.concatenate(yns, axis=0)
    final = _k6(yn.reshape(B * S, D), W1.astype(jnp.bfloat16), b1,
                W2.astype(jnp.bfloat16), x2.reshape(B * S, D), b2)
    return final.reshape(B, S, D)


# speedup vs baseline: 7.6043x; 1.0033x over previous
import functools

import jax
import jax.numpy as jnp
from jax import lax
from jax.experimental import pallas as pl
from jax.experimental.pallas import tpu as pltpu
from jax.experimental.pallas import tpu_sc as plsc

B = 2
S = 4096
D = 1024
H = 16
HD = 64
MLP_D = 4096
CHUNK = 128
NB = 64
P = B * H
NCHUNK = S // CHUNK

NC = 2
NS = 16
LANES = 16
STRIDE = S // LANES
GR = 256


def _k1_body(x_ref, s1_ref, b1_ref, wqk_ref, wv_ref, rot_ref,
             qkv_ref, bkt_ref, rn_ref):
    x = x_ref[0]
    mu = jnp.mean(x, axis=-1, keepdims=True)
    var = jnp.mean(jnp.square(x - mu), axis=-1, keepdims=True)
    xn = (x - mu) * lax.rsqrt(var + 1e-6) * s1_ref[...] + b1_ref[...]
    qk = jnp.dot(xn, wqk_ref[...], preferred_element_type=jnp.float32)
    v = jnp.dot(xn.astype(jnp.bfloat16), wv_ref[...],
                preferred_element_type=jnp.float32)
    bs = x.shape[0]
    rall = jnp.dot(qk, rot_ref[...], preferred_element_type=jnp.float32)
    aabs = jnp.abs(rall)
    codes = (lax.broadcasted_iota(jnp.int32, (bs, H * NB // 2), 1) % (NB // 2)
             + jnp.where(rall < 0.0, NB // 2, 0))
    for h in range(H):
        qh = qk[:, h * HD:(h + 1) * HD]
        qkv_ref[0, h, :, 0:HD] = qh
        qkv_ref[0, h, :, HD:2 * HD] = v[:, h * HD:(h + 1) * HD]
        ss = jnp.sum(jnp.square(qh), axis=-1, keepdims=True)
        rn_ref[0, :, h:h + 1] = 0.125 / (jnp.sqrt(ss) + 1e-6)
        a = aabs[:, h * (NB // 2):(h + 1) * (NB // 2)]
        m = jnp.max(a, axis=-1, keepdims=True)
        cand = jnp.where(a == m, codes[:, h * (NB // 2):(h + 1) * (NB // 2)],
                         NB)
        bkt_ref[0, :, h:h + 1] = jnp.min(cand, axis=-1, keepdims=True)


def _k1(x, s1, b1, wqk2, wv2, rot):
    bs = 512
    nb = x.shape[0]
    grid = (nb, S // bs)
    return pl.pallas_call(
        _k1_body,
        grid=grid,
        in_specs=[
            pl.BlockSpec((1, bs, D), lambda b, s: (b, s, 0)),
            pl.BlockSpec((D,), lambda b, s: (0,)),
            pl.BlockSpec((D,), lambda b, s: (0,)),
            pl.BlockSpec((D, H * HD), lambda b, s: (0, 0)),
            pl.BlockSpec((D, H * HD), lambda b, s: (0, 0)),
            pl.BlockSpec((H * HD, H * NB // 2), lambda b, s: (0, 0)),
        ],
        out_specs=[
            pl.BlockSpec((1, H, bs, 2 * HD), lambda b, s: (b, 0, s, 0)),
            pl.BlockSpec((1, bs, H), lambda b, s: (b, s, 0)),
            pl.BlockSpec((1, bs, H), lambda b, s: (b, s, 0)),
        ],
        out_shape=[
            jax.ShapeDtypeStruct((nb, H, S, 2 * HD), jnp.float32),
            jax.ShapeDtypeStruct((nb, S, H), jnp.int32),
            jax.ShapeDtypeStruct((nb, S, H), jnp.float32),
        ],
    )(x, s1, b1, wqk2, wv2, rot)


def _sc_sort_body(bkt_hbm, rn_hbm, qkv_hbm,
                  skey_hbm, undo_hbm, srn_hbm, sqkv_hbm,
                  bkt_v, cnt_v, tot_v, off_v, lr_v, sidx_v, undo_v, skey_v,
                  rn_v, srn_v, buf, buf2, sem, sem2):
    w = lax.axis_index("s")
    half = lax.axis_index("c")
    iot = lax.iota(jnp.int32, LANES)
    pltpu.sync_copy(bkt_hbm.at[w], bkt_v)
    pltpu.sync_copy(rn_hbm.at[w], rn_v)

    def zero(i, _):
        cnt_v[pl.ds(i * LANES, LANES)] = jnp.zeros((LANES,), jnp.int32)
        return 0
    lax.fori_loop(0, NB * LANES // LANES, zero, 0)

    def p1(t, _):
        idxv = iot * STRIDE + t
        bk = plsc.load_gather(bkt_v, [idxv])
        cidx = iot * NB + bk
        c = plsc.load_gather(cnt_v, [cidx])
        plsc.store_scatter(lr_v, [idxv], c)
        plsc.store_scatter(cnt_v, [cidx], c + 1)
        return 0
    lax.fori_loop(0, STRIDE, p1, 0)

    def p2(bkt, _):
        colidx = iot * NB + bkt
        c = plsc.load_gather(cnt_v, [colidx])
        s = plsc.cumsum(c)
        plsc.store_scatter(cnt_v, [colidx], s - c)
        plsc.store_scatter(tot_v, [iot * 0 + bkt], s, mask=iot == LANES - 1)
        return 0
    lax.fori_loop(0, NB, p2, 0)

    def p3(g, carry):
        tv = tot_v[pl.ds(g * LANES, LANES)]
        s = plsc.cumsum(tv)
        off_v[pl.ds(g * LANES, LANES)] = s - tv + carry
        return carry + jnp.sum(tv)
    lax.fori_loop(0, NB // LANES, p3, jnp.int32(0))

    def p4(t, _):
        idxv = iot * STRIDE + t
        bk = plsc.load_gather(bkt_v, [idxv])
        lr = plsc.load_gather(lr_v, [idxv])
        sp = plsc.load_gather(cnt_v, [iot * NB + bk])
        ob = plsc.load_gather(off_v, [bk])
        pos = ob + sp + lr
        plsc.store_scatter(sidx_v, [pos], idxv)
        plsc.store_scatter(undo_v, [idxv], pos)
        plsc.store_scatter(skey_v, [pos], bk * S + idxv)
        rv = plsc.load_gather(rn_v, [idxv])
        plsc.store_scatter(srn_v, [pos], rv)
        return 0
    lax.fori_loop(0, STRIDE, p4, 0)

    @pl.when(half == 0)
    def _():
        pltpu.sync_copy(skey_v, skey_hbm.at[w])
        pltpu.sync_copy(undo_v, undo_hbm.at[w])
        pltpu.sync_copy(srn_v, srn_hbm.at[w])

    _pipelined_gather(qkv_hbm.at[w], sidx_v, sqkv_hbm.at[w], half,
                      (buf, buf2), (sem, sem2))


def _pipelined_gather(table, idx_v, out, half, bufs, sems):
    nch = S // GR // 2
    base = half * nch
    cps = [None, None]
    for j in range(nch):
        cps[j % 2] = pltpu.async_copy(
            table.at[idx_v.at[pl.ds((base + j) * GR, GR)]],
            bufs[j % 2], sems[j % 2])
        if j >= 1:
            cps[(j - 1) % 2].wait()
            pltpu.sync_copy(bufs[(j - 1) % 2],
                            out.at[pl.ds((base + j - 1) * GR, GR)])
    cps[(nch - 1) % 2].wait()
    pltpu.sync_copy(bufs[(nch - 1) % 2],
                    out.at[pl.ds((base + nch - 1) * GR, GR)])


def _k2(bkt, rn, qkv):
    mesh = plsc.VectorSubcoreMesh(core_axis_name="c", subcore_axis_name="s",
                                  num_cores=NC)
    f = functools.partial(
        pl.kernel,
        out_type=(
            jax.ShapeDtypeStruct((H, S), jnp.int32),
            jax.ShapeDtypeStruct((H, S), jnp.int32),
            jax.ShapeDtypeStruct((H, S), jnp.float32),
            jax.ShapeDtypeStruct((H, S, 2 * HD), jnp.float32),
        ),
        mesh=mesh,
        scratch_types=[
            pltpu.VMEM((S,), jnp.int32),
            pltpu.VMEM((NB * LANES,), jnp.int32),
            pltpu.VMEM((NB,), jnp.int32),
            pltpu.VMEM((NB,), jnp.int32),
            pltpu.VMEM((S,), jnp.int32),
            pltpu.VMEM((S,), jnp.int32),
            pltpu.VMEM((S,), jnp.int32),
            pltpu.VMEM((S,), jnp.int32),
            pltpu.VMEM((S,), jnp.float32),
            pltpu.VMEM((S,), jnp.float32),
            pltpu.VMEM((GR, 2 * HD), jnp.float32),
            pltpu.VMEM((GR, 2 * HD), jnp.float32),
            pltpu.SemaphoreType.DMA,
            pltpu.SemaphoreType.DMA,
        ],
        compiler_params=pltpu.CompilerParams(needs_layout_passes=False),
    )(_sc_sort_body)
    return f(bkt, rn, qkv)


def _sc_unsort_body(undo_hbm, os_hbm, attn_hbm, undo_v, buf, buf2, sem, sem2):
    w = lax.axis_index("s")
    half = lax.axis_index("c")
    pltpu.sync_copy(undo_hbm.at[w], undo_v)
    _pipelined_gather(os_hbm.at[w], undo_v, attn_hbm.at[w], half,
                      (buf, buf2), (sem, sem2))


def _k4(undo, osort):
    mesh = plsc.VectorSubcoreMesh(core_axis_name="c", subcore_axis_name="s",
                                  num_cores=NC)
    f = functools.partial(
        pl.kernel,
        out_type=jax.ShapeDtypeStruct((H, S, 2 * HD), jnp.float32),
        mesh=mesh,
        scratch_types=[
            pltpu.VMEM((S,), jnp.int32),
            pltpu.VMEM((GR, 2 * HD), jnp.float32),
            pltpu.VMEM((GR, 2 * HD), jnp.float32),
            pltpu.SemaphoreType.DMA,
            pltpu.SemaphoreType.DMA,
        ],
        compiler_params=pltpu.CompilerParams(needs_layout_passes=False),
    )(_sc_unsort_body)
    return f(undo, osort)


CPB = 16
KBS = CPB * CHUNK
NSB = S // KBS


def _k3_body(sb_ref, pb_ref, nb_ref, ts_ref, tp_ref, tn_ref,
             rs_ref, rp_ref, rn_ref, o_ref):
    sblk = sb_ref[0]
    tsel = ts_ref[0, 0]
    rsel = rs_ref[0, 0]
    outs = []
    for i in range(CPB):
        lo = i * CHUNK
        qbf = sblk[lo:lo + CHUNK, :HD].astype(jnp.bfloat16)
        qb = (tsel[lo:lo + CHUNK] // S)[:, None]
        if i == 0:
            pieces = [
                (pb_ref[0], tp_ref[0, 0, (CPB - 1) * CHUNK:],
                 rp_ref[0, 0, (CPB - 1) * CHUNK:], None),
                (sblk[0:2 * CHUNK, :], tsel[0:2 * CHUNK],
                 rsel[0:2 * CHUNK], 0),
            ]
        elif i == CPB - 1:
            pieces = [
                (sblk[lo - CHUNK:lo + CHUNK, :], tsel[lo - CHUNK:lo + CHUNK],
                 rsel[lo - CHUNK:lo + CHUNK], CHUNK),
                (nb_ref[0], tn_ref[0, 0, :CHUNK], rn_ref[0, 0, :CHUNK], None),
            ]
        else:
            pieces = [
                (sblk[lo - CHUNK:lo + 2 * CHUNK, :],
                 tsel[lo - CHUNK:lo + 2 * CHUNK],
                 rsel[lo - CHUNK:lo + 2 * CHUNK], CHUNK),
            ]
        ds, vs = [], []
        for kv, tk, rk, doff in pieces:
            wdt = kv.shape[0]
            d = jax.lax.dot_general(
                qbf, kv[:, :HD].astype(jnp.bfloat16),
                (((1,), (1,)), ((), ())),
                preferred_element_type=jnp.float32) * rk[None, :]
            d = jnp.where(qb != (tk // S)[None, :], d - 1e9, d)
            if doff is not None:
                eye = (lax.broadcasted_iota(jnp.int32, (CHUNK, wdt), 1)
                       == lax.broadcasted_iota(jnp.int32, (CHUNK, wdt), 0)
                       + doff)
                d = jnp.where(eye, d - 1e5, d)
            ds.append(d)
            vs.append(kv[:, HD:].astype(jnp.bfloat16))
        if len(ds) == 1:
            m = jnp.max(ds[0], axis=-1, keepdims=True)
            e = jnp.exp(ds[0] - m)
            tot = jnp.sum(e, axis=-1, keepdims=True)
            o = jnp.dot(e.astype(jnp.bfloat16), vs[0],
                        preferred_element_type=jnp.float32)
        else:
            m = jnp.maximum(jnp.max(ds[0], axis=-1, keepdims=True),
                            jnp.max(ds[1], axis=-1, keepdims=True))
            e0 = jnp.exp(ds[0] - m)
            e1 = jnp.exp(ds[1] - m)
            tot = (jnp.sum(e0, axis=-1, keepdims=True)
                   + jnp.sum(e1, axis=-1, keepdims=True))
            o = (jnp.dot(e0.astype(jnp.bfloat16), vs[0],
                         preferred_element_type=jnp.float32)
                 + jnp.dot(e1.astype(jnp.bfloat16), vs[1],
                           preferred_element_type=jnp.float32))
        o = o * (1.0 / tot)
        outs.append(jnp.concatenate([o, jnp.zeros_like(o)], axis=1))
    o_ref[0] = jnp.concatenate(outs, axis=0)


def _k3(sqkv, skey3, srn3):
    def prev(p, c):
        return (p, (c * CPB + NCHUNK - 1) % NCHUNK, 0)

    def nxt(p, c):
        return (p, (c * CPB + CPB) % NCHUNK, 0)

    def tprev(p, c):
        return (p * NSB + (c + NSB - 1) % NSB, 0, 0)

    def tnxt(p, c):
        return (p * NSB + (c + 1) % NSB, 0, 0)

    def tself(p, c):
        return (p * NSB + c, 0, 0)

    return pl.pallas_call(
        _k3_body,
        grid=(sqkv.shape[0], NSB),
        in_specs=[
            pl.BlockSpec((1, KBS, 2 * HD), lambda p, c: (p, c, 0)),
            pl.BlockSpec((1, CHUNK, 2 * HD), prev),
            pl.BlockSpec((1, CHUNK, 2 * HD), nxt),
            pl.BlockSpec((1, 1, KBS), tself),
            pl.BlockSpec((1, 1, KBS), tprev),
            pl.BlockSpec((1, 1, KBS), tnxt),
            pl.BlockSpec((1, 1, KBS), tself),
            pl.BlockSpec((1, 1, KBS), tprev),
            pl.BlockSpec((1, 1, KBS), tnxt),
        ],
        out_specs=pl.BlockSpec((1, KBS, 2 * HD), lambda p, c: (p, c, 0)),
        out_shape=jax.ShapeDtypeStruct((sqkv.shape[0], S, 2 * HD),
                                       jnp.float32),
    )(sqkv, sqkv, sqkv, skey3, skey3, skey3, srn3, srn3, srn3)


def _k56_body(a_ref, x_ref, wo_ref, s2_ref, b2w_ref, w1_ref, b1_ref, w2_ref,
              b2_ref, out_ref, acc_ref, yn_ref):
    n = pl.program_id(1)
    nblocks = pl.num_programs(1)

    @pl.when(n == 0)
    def _():
        cat = jnp.concatenate([a_ref[h, :, :HD] for h in range(H)], axis=1)
        out = jnp.dot(cat.astype(jnp.bfloat16), wo_ref[...],
                      preferred_element_type=jnp.float32)
        x2 = x_ref[0] + out
        mu = jnp.mean(x2, axis=-1, keepdims=True)
        var = jnp.mean(jnp.square(x2 - mu), axis=-1, keepdims=True)
        yn = (x2 - mu) * lax.rsqrt(var + 1e-6) * s2_ref[...] + b2w_ref[...]
        yn_ref[...] = yn.astype(jnp.bfloat16)
        acc_ref[...] = x2 + b2_ref[...]

    hid = jnp.dot(yn_ref[...], w1_ref[...],
                  preferred_element_type=jnp.float32) + b1_ref[...]
    hid = jnp.maximum(hid, 0.0).astype(jnp.bfloat16)
    acc_ref[...] += jnp.dot(hid, w2_ref[...],
                            preferred_element_type=jnp.float32)

    @pl.when(n == nblocks - 1)
    def _():
        out_ref[0] = acc_ref[...]


def _k56(attn, x, wo2, s2, b2w, w1, b1, w2, b2):
    rb = 512
    nb = 1024
    return pl.pallas_call(
        _k56_body,
        grid=(S // rb, MLP_D // nb),
        in_specs=[
            pl.BlockSpec((H, rb, 2 * HD), lambda r, n: (0, r, 0)),
            pl.BlockSpec((1, rb, D), lambda r, n: (0, r, 0)),
            pl.BlockSpec((H * HD, D), lambda r, n: (0, 0)),
            pl.BlockSpec((D,), lambda r, n: (0,)),
            pl.BlockSpec((D,), lambda r, n: (0,)),
            pl.BlockSpec((D, nb), lambda r, n: (0, n)),
            pl.BlockSpec((nb,), lambda r, n: (n,)),
            pl.BlockSpec((nb, D), lambda r, n: (n, 0)),
            pl.BlockSpec((D,), lambda r, n: (0,)),
        ],
        out_specs=pl.BlockSpec((1, rb, D), lambda r, n: (0, r, 0)),
        out_shape=jax.ShapeDtypeStruct((1, S, D), jnp.float32),
        scratch_shapes=[pltpu.VMEM((rb, D), jnp.float32),
                        pltpu.VMEM((rb, D), jnp.bfloat16)],
    )(attn, x, wo2, s2, b2w, w1, b1, w2, b2)


def kernel(inputs, ln1_scale, ln1_bias, Wqk, Wv, Wo, rot, ln2_scale,
           ln2_bias, W1, b1, W2, b2):
    wqk2 = Wqk.reshape(D, H * HD)
    wv2 = Wv.reshape(D, H * HD).astype(jnp.bfloat16)
    wo2 = Wo.reshape(H * HD, D).astype(jnp.bfloat16)
    rotbd = jax.scipy.linalg.block_diag(*[rot[h] for h in range(H)])

    w1b = W1.astype(jnp.bfloat16)
    w2b = W2.astype(jnp.bfloat16)
    finals = []
    for b in range(B):
        xb = lax.slice_in_dim(inputs, b, b + 1, axis=0)
        qkv4, bkt3, rn4 = _k1(xb, ln1_scale, ln1_bias, wqk2, wv2, rotbd)
        qkv = qkv4.reshape(H, S, 2 * HD)
        bkt = jnp.transpose(bkt3, (0, 2, 1)).reshape(H, S)
        rn = jnp.transpose(rn4, (0, 2, 1)).reshape(H, S)

        skey, undo, srn, sqkv = _k2(bkt, rn, qkv)
        skey3 = skey.reshape(H * NSB, 1, KBS)
        srn3 = srn.reshape(H * NSB, 1, KBS)

        osort = _k3(sqkv, skey3, srn3)
        attn = _k4(undo, osort)

        finals.append(_k56(attn, xb, wo2, ln2_scale, ln2_bias,
                           w1b, b1, w2b, b2))
    return jnp.concatenate(finals, axis=0)


# speedup vs baseline: 7.7203x; 1.0153x over previous
import functools

import jax
import jax.numpy as jnp
from jax import lax
from jax.experimental import pallas as pl
from jax.experimental.pallas import tpu as pltpu
from jax.experimental.pallas import tpu_sc as plsc

B = 2
S = 4096
D = 1024
H = 16
HD = 64
MLP_D = 4096
CHUNK = 128
NB = 64
P = B * H
NCHUNK = S // CHUNK

NC = 2
NS = 16
LANES = 16
STRIDE = S // LANES
GR = 256


def _k1_body(x_ref, s1_ref, b1_ref, wqk_ref, wv_ref, rot_ref,
             qkv_ref, bkt_ref, rn_ref):
    x = x_ref[0]
    mu = jnp.mean(x, axis=-1, keepdims=True)
    var = jnp.mean(jnp.square(x - mu), axis=-1, keepdims=True)
    xn = (x - mu) * lax.rsqrt(var + 1e-6) * s1_ref[...] + b1_ref[...]
    qk = jnp.dot(xn, wqk_ref[...], preferred_element_type=jnp.float32)
    v = jnp.dot(xn.astype(jnp.bfloat16), wv_ref[...],
                preferred_element_type=jnp.float32)
    bs = x.shape[0]
    rall = jnp.dot(qk, rot_ref[...], preferred_element_type=jnp.float32)
    aabs = jnp.abs(rall)
    codes = (lax.broadcasted_iota(jnp.int32, (bs, H * NB // 2), 1) % (NB // 2)
             + jnp.where(rall < 0.0, NB // 2, 0))
    for h in range(H):
        qh = qk[:, h * HD:(h + 1) * HD]
        qkv_ref[0, h, :, 0:HD] = qh
        qkv_ref[0, h, :, HD:2 * HD] = v[:, h * HD:(h + 1) * HD]
        ss = jnp.sum(jnp.square(qh), axis=-1, keepdims=True)
        rn_ref[0, :, h:h + 1] = 0.125 / (jnp.sqrt(ss) + 1e-6)
        a = aabs[:, h * (NB // 2):(h + 1) * (NB // 2)]
        m = jnp.max(a, axis=-1, keepdims=True)
        cand = jnp.where(a == m, codes[:, h * (NB // 2):(h + 1) * (NB // 2)],
                         NB)
        bkt_ref[0, :, h:h + 1] = jnp.min(cand, axis=-1, keepdims=True)


def _k1(x, s1, b1, wqk2, wv2, rot):
    bs = 512
    nb = x.shape[0]
    grid = (nb, S // bs)
    return pl.pallas_call(
        _k1_body,
        grid=grid,
        in_specs=[
            pl.BlockSpec((1, bs, D), lambda b, s: (b, s, 0)),
            pl.BlockSpec((D,), lambda b, s: (0,)),
            pl.BlockSpec((D,), lambda b, s: (0,)),
            pl.BlockSpec((D, H * HD), lambda b, s: (0, 0)),
            pl.BlockSpec((D, H * HD), lambda b, s: (0, 0)),
            pl.BlockSpec((H * HD, H * NB // 2), lambda b, s: (0, 0)),
        ],
        out_specs=[
            pl.BlockSpec((1, H, bs, 2 * HD), lambda b, s: (b, 0, s, 0)),
            pl.BlockSpec((1, bs, H), lambda b, s: (b, s, 0)),
            pl.BlockSpec((1, bs, H), lambda b, s: (b, s, 0)),
        ],
        out_shape=[
            jax.ShapeDtypeStruct((nb, H, S, 2 * HD), jnp.float32),
            jax.ShapeDtypeStruct((nb, S, H), jnp.int32),
            jax.ShapeDtypeStruct((nb, S, H), jnp.float32),
        ],
    )(x, s1, b1, wqk2, wv2, rot)


def _sc_sort_body(bkt_hbm, rn_hbm, qkv_hbm,
                  skey_hbm, undo_hbm, srn_hbm, sqkv_hbm,
                  bkt_v, cnt_v, tot_v, off_v, lr_v, sidx_v, undo_v, skey_v,
                  rn_v, srn_v, buf, buf2, sem, sem2):
    w = lax.axis_index("s")
    half = lax.axis_index("c")
    iot = lax.iota(jnp.int32, LANES)
    pltpu.sync_copy(bkt_hbm.at[w], bkt_v)
    pltpu.sync_copy(rn_hbm.at[w], rn_v)

    def zero(i, _):
        cnt_v[pl.ds(i * LANES, LANES)] = jnp.zeros((LANES,), jnp.int32)
        return 0
    lax.fori_loop(0, NB * LANES // LANES, zero, 0)

    def p1(t, _):
        idxv = iot * STRIDE + t
        bk = plsc.load_gather(bkt_v, [idxv])
        cidx = iot * NB + bk
        c = plsc.load_gather(cnt_v, [cidx])
        plsc.store_scatter(lr_v, [idxv], c)
        plsc.store_scatter(cnt_v, [cidx], c + 1)
        return 0
    lax.fori_loop(0, STRIDE, p1, 0)

    def p2(bkt, _):
        colidx = iot * NB + bkt
        c = plsc.load_gather(cnt_v, [colidx])
        s = plsc.cumsum(c)
        plsc.store_scatter(cnt_v, [colidx], s - c)
        plsc.store_scatter(tot_v, [iot * 0 + bkt], s, mask=iot == LANES - 1)
        return 0
    lax.fori_loop(0, NB, p2, 0)

    def p3(g, carry):
        tv = tot_v[pl.ds(g * LANES, LANES)]
        s = plsc.cumsum(tv)
        off_v[pl.ds(g * LANES, LANES)] = s - tv + carry
        return carry + jnp.sum(tv)
    lax.fori_loop(0, NB // LANES, p3, jnp.int32(0))

    def p4(t, _):
        idxv = iot * STRIDE + t
        bk = plsc.load_gather(bkt_v, [idxv])
        lr = plsc.load_gather(lr_v, [idxv])
        sp = plsc.load_gather(cnt_v, [iot * NB + bk])
        ob = plsc.load_gather(off_v, [bk])
        pos = ob + sp + lr
        plsc.store_scatter(sidx_v, [pos], idxv)
        plsc.store_scatter(undo_v, [idxv], pos)
        plsc.store_scatter(skey_v, [pos], bk * S + idxv)
        rv = plsc.load_gather(rn_v, [idxv])
        plsc.store_scatter(srn_v, [pos], rv)
        return 0
    lax.fori_loop(0, STRIDE, p4, 0)

    @pl.when(half == 0)
    def _():
        pltpu.sync_copy(skey_v, skey_hbm.at[w])
        pltpu.sync_copy(undo_v, undo_hbm.at[w])
        pltpu.sync_copy(srn_v, srn_hbm.at[w])

    _pipelined_gather(qkv_hbm.at[w], sidx_v, sqkv_hbm.at[w], half,
                      (buf, buf2), (sem, sem2))


def _pipelined_gather(table, idx_v, out, half, bufs, sems):
    nch = S // GR // 2
    base = half * nch
    cps = [None, None]
    for j in range(nch):
        cps[j % 2] = pltpu.async_copy(
            table.at[idx_v.at[pl.ds((base + j) * GR, GR)]],
            bufs[j % 2], sems[j % 2])
        if j >= 1:
            cps[(j - 1) % 2].wait()
            pltpu.sync_copy(bufs[(j - 1) % 2],
                            out.at[pl.ds((base + j - 1) * GR, GR)])
    cps[(nch - 1) % 2].wait()
    pltpu.sync_copy(bufs[(nch - 1) % 2],
                    out.at[pl.ds((base + nch - 1) * GR, GR)])


def _k2(bkt, rn, qkv):
    mesh = plsc.VectorSubcoreMesh(core_axis_name="c", subcore_axis_name="s",
                                  num_cores=NC)
    f = functools.partial(
        pl.kernel,
        out_type=(
            jax.ShapeDtypeStruct((H, S), jnp.int32),
            jax.ShapeDtypeStruct((H, S), jnp.int32),
            jax.ShapeDtypeStruct((H, S), jnp.float32),
            jax.ShapeDtypeStruct((H, S, 2 * HD), jnp.float32),
        ),
        mesh=mesh,
        scratch_types=[
            pltpu.VMEM((S,), jnp.int32),
            pltpu.VMEM((NB * LANES,), jnp.int32),
            pltpu.VMEM((NB,), jnp.int32),
            pltpu.VMEM((NB,), jnp.int32),
            pltpu.VMEM((S,), jnp.int32),
            pltpu.VMEM((S,), jnp.int32),
            pltpu.VMEM((S,), jnp.int32),
            pltpu.VMEM((S,), jnp.int32),
            pltpu.VMEM((S,), jnp.float32),
            pltpu.VMEM((S,), jnp.float32),
            pltpu.VMEM((GR, 2 * HD), jnp.float32),
            pltpu.VMEM((GR, 2 * HD), jnp.float32),
            pltpu.SemaphoreType.DMA,
            pltpu.SemaphoreType.DMA,
        ],
        compiler_params=pltpu.CompilerParams(needs_layout_passes=False),
    )(_sc_sort_body)
    return f(bkt, rn, qkv)


def _sc_unsort_body(undo_hbm, os_hbm, attn_hbm, undo_v, buf, buf2, sem, sem2):
    w = lax.axis_index("s")
    half = lax.axis_index("c")
    pltpu.sync_copy(undo_hbm.at[w], undo_v)
    _pipelined_gather(os_hbm.at[w], undo_v, attn_hbm.at[w], half,
                      (buf, buf2), (sem, sem2))


def _k4(undo, osort):
    mesh = plsc.VectorSubcoreMesh(core_axis_name="c", subcore_axis_name="s",
                                  num_cores=NC)
    f = functools.partial(
        pl.kernel,
        out_type=jax.ShapeDtypeStruct((H, S, 2 * HD), jnp.float32),
        mesh=mesh,
        scratch_types=[
            pltpu.VMEM((S,), jnp.int32),
            pltpu.VMEM((GR, 2 * HD), jnp.float32),
            pltpu.VMEM((GR, 2 * HD), jnp.float32),
            pltpu.SemaphoreType.DMA,
            pltpu.SemaphoreType.DMA,
        ],
        compiler_params=pltpu.CompilerParams(needs_layout_passes=False),
    )(_sc_unsort_body)
    return f(undo, osort)


CPB = 32
KBS = CPB * CHUNK
NSB = S // KBS


def _k3_body(sb_ref, ts_ref, rs_ref, o_ref):
    sblk = sb_ref[0]
    tsel = ts_ref[0, 0]
    rsel = rs_ref[0, 0]
    outs = []
    for i in range(CPB):
        lo = i * CHUNK
        qbf = sblk[lo:lo + CHUNK, :HD].astype(jnp.bfloat16)
        qb = (tsel[lo:lo + CHUNK] // S)[:, None]
        if i == 0:
            pieces = [
                (sblk[(CPB - 1) * CHUNK:, :], tsel[(CPB - 1) * CHUNK:],
                 rsel[(CPB - 1) * CHUNK:], None),
                (sblk[0:2 * CHUNK, :], tsel[0:2 * CHUNK],
                 rsel[0:2 * CHUNK], 0),
            ]
        elif i == CPB - 1:
            pieces = [
                (sblk[lo - CHUNK:lo + CHUNK, :], tsel[lo - CHUNK:lo + CHUNK],
                 rsel[lo - CHUNK:lo + CHUNK], CHUNK),
                (sblk[0:CHUNK, :], tsel[0:CHUNK], rsel[0:CHUNK], None),
            ]
        else:
            pieces = [
                (sblk[lo - CHUNK:lo + 2 * CHUNK, :],
                 tsel[lo - CHUNK:lo + 2 * CHUNK],
                 rsel[lo - CHUNK:lo + 2 * CHUNK], CHUNK),
            ]
        ds, vs = [], []
        for kv, tk, rk, doff in pieces:
            wdt = kv.shape[0]
            d = jax.lax.dot_general(
                qbf, kv[:, :HD].astype(jnp.bfloat16),
                (((1,), (1,)), ((), ())),
                preferred_element_type=jnp.float32) * rk[None, :]
            d = jnp.where(qb != (tk // S)[None, :], d - 1e9, d)
            if doff is not None:
                eye = (lax.broadcasted_iota(jnp.int32, (CHUNK, wdt), 1)
                       == lax.broadcasted_iota(jnp.int32, (CHUNK, wdt), 0)
                       + doff)
                d = jnp.where(eye, d - 1e5, d)
            ds.append(d)
            vs.append(kv[:, HD:].astype(jnp.bfloat16))
        if len(ds) == 1:
            m = jnp.max(ds[0], axis=-1, keepdims=True)
            e = jnp.exp(ds[0] - m)
            tot = jnp.sum(e, axis=-1, keepdims=True)
            o = jnp.dot(e.astype(jnp.bfloat16), vs[0],
                        preferred_element_type=jnp.float32)
        else:
            m = jnp.maximum(jnp.max(ds[0], axis=-1, keepdims=True),
                            jnp.max(ds[1], axis=-1, keepdims=True))
            e0 = jnp.exp(ds[0] - m)
            e1 = jnp.exp(ds[1] - m)
            tot = (jnp.sum(e0, axis=-1, keepdims=True)
                   + jnp.sum(e1, axis=-1, keepdims=True))
            o = (jnp.dot(e0.astype(jnp.bfloat16), vs[0],
                         preferred_element_type=jnp.float32)
                 + jnp.dot(e1.astype(jnp.bfloat16), vs[1],
                           preferred_element_type=jnp.float32))
        o = o * (1.0 / tot)
        outs.append(jnp.concatenate([o, jnp.zeros_like(o)], axis=1))
    o_ref[0] = jnp.concatenate(outs, axis=0)


def _k3(sqkv, skey3, srn3):
    return pl.pallas_call(
        _k3_body,
        grid=(sqkv.shape[0],),
        in_specs=[
            pl.BlockSpec((1, KBS, 2 * HD), lambda p: (p, 0, 0)),
            pl.BlockSpec((1, 1, KBS), lambda p: (p, 0, 0)),
            pl.BlockSpec((1, 1, KBS), lambda p: (p, 0, 0)),
        ],
        out_specs=pl.BlockSpec((1, KBS, 2 * HD), lambda p: (p, 0, 0)),
        out_shape=jax.ShapeDtypeStruct((sqkv.shape[0], S, 2 * HD),
                                       jnp.float32),
    )(sqkv, skey3, srn3)


def _k56_body(a_ref, x_ref, wo_ref, s2_ref, b2w_ref, w1_ref, b1_ref, w2_ref,
              b2_ref, out_ref, acc_ref, yn_ref):
    n = pl.program_id(1)
    nblocks = pl.num_programs(1)

    @pl.when(n == 0)
    def _():
        cat = jnp.concatenate([a_ref[h, :, :HD] for h in range(H)], axis=1)
        out = jnp.dot(cat.astype(jnp.bfloat16), wo_ref[...],
                      preferred_element_type=jnp.float32)
        x2 = x_ref[0] + out
        mu = jnp.mean(x2, axis=-1, keepdims=True)
        var = jnp.mean(jnp.square(x2 - mu), axis=-1, keepdims=True)
        yn = (x2 - mu) * lax.rsqrt(var + 1e-6) * s2_ref[...] + b2w_ref[...]
        yn_ref[...] = yn.astype(jnp.bfloat16)
        acc_ref[...] = x2 + b2_ref[...]

    hid = jnp.dot(yn_ref[...], w1_ref[...],
                  preferred_element_type=jnp.float32) + b1_ref[...]
    hid = jnp.maximum(hid, 0.0).astype(jnp.bfloat16)
    acc_ref[...] += jnp.dot(hid, w2_ref[...],
                            preferred_element_type=jnp.float32)

    @pl.when(n == nblocks - 1)
    def _():
        out_ref[0] = acc_ref[...]


def _k56(attn, x, wo2, s2, b2w, w1, b1, w2, b2):
    rb = 512
    nb = 1024
    return pl.pallas_call(
        _k56_body,
        grid=(S // rb, MLP_D // nb),
        in_specs=[
            pl.BlockSpec((H, rb, 2 * HD), lambda r, n: (0, r, 0)),
            pl.BlockSpec((1, rb, D), lambda r, n: (0, r, 0)),
            pl.BlockSpec((H * HD, D), lambda r, n: (0, 0)),
            pl.BlockSpec((D,), lambda r, n: (0,)),
            pl.BlockSpec((D,), lambda r, n: (0,)),
            pl.BlockSpec((D, nb), lambda r, n: (0, n)),
            pl.BlockSpec((nb,), lambda r, n: (n,)),
            pl.BlockSpec((nb, D), lambda r, n: (n, 0)),
            pl.BlockSpec((D,), lambda r, n: (0,)),
        ],
        out_specs=pl.BlockSpec((1, rb, D), lambda r, n: (0, r, 0)),
        out_shape=jax.ShapeDtypeStruct((1, S, D), jnp.float32),
        scratch_shapes=[pltpu.VMEM((rb, D), jnp.float32),
                        pltpu.VMEM((rb, D), jnp.bfloat16)],
    )(attn, x, wo2, s2, b2w, w1, b1, w2, b2)


def kernel(inputs, ln1_scale, ln1_bias, Wqk, Wv, Wo, rot, ln2_scale,
           ln2_bias, W1, b1, W2, b2):
    wqk2 = Wqk.reshape(D, H * HD)
    wv2 = Wv.reshape(D, H * HD).astype(jnp.bfloat16)
    wo2 = Wo.reshape(H * HD, D).astype(jnp.bfloat16)
    rotbd = jax.scipy.linalg.block_diag(*[rot[h] for h in range(H)])

    w1b = W1.astype(jnp.bfloat16)
    w2b = W2.astype(jnp.bfloat16)
    finals = []
    for b in range(B):
        xb = lax.slice_in_dim(inputs, b, b + 1, axis=0)
        qkv4, bkt3, rn4 = _k1(xb, ln1_scale, ln1_bias, wqk2, wv2, rotbd)
        qkv = qkv4.reshape(H, S, 2 * HD)
        bkt = jnp.transpose(bkt3, (0, 2, 1)).reshape(H, S)
        rn = jnp.transpose(rn4, (0, 2, 1)).reshape(H, S)

        skey, undo, srn, sqkv = _k2(bkt, rn, qkv)
        skey3 = skey.reshape(H * NSB, 1, KBS)
        srn3 = srn.reshape(H * NSB, 1, KBS)

        osort = _k3(sqkv, skey3, srn3)
        attn = _k4(undo, osort)

        finals.append(_k56(attn, xb, wo2, ln2_scale, ln2_bias,
                           w1b, b1, w2b, b2))
    return jnp.concatenate(finals, axis=0)
